# Initial kernel scaffold; baseline (speedup 1.0000x reference)
import functools

import jax
import jax.numpy as jnp
from jax import lax
from jax.experimental import pallas as pl
from jax.experimental.pallas import tpu as pltpu
from jax.experimental.pallas import tpu_sc as plsc

NC = 2
NS = 16
NW = NC * NS

_f32 = jnp.float32


def _make_edge_pass(NP, D, CPT):
  RPT = NP // NS
  mesh = plsc.VectorSubcoreMesh(core_axis_name="c", subcore_axis_name="s")

  @functools.partial(
      pl.kernel,
      out_type=(
          jax.ShapeDtypeStruct((NC * NP, D), _f32),
          jax.ShapeDtypeStruct((NC * NP,), _f32),
      ),
      mesh=mesh,
      scratch_types=[
          pltpu.VMEM((CPT, 128), jnp.int32),
          pltpu.VMEM((CPT, 128), jnp.int32),
          pltpu.VMEM((128, D), _f32),
          pltpu.VMEM((128, D), _f32),
          pltpu.VMEM((128,), _f32),
          pltpu.VMEM((NP // NS,), _f32),
          pltpu.VMEM_SHARED((NP, D), _f32),
          pltpu.VMEM_SHARED((NP,), _f32),
          pltpu.SemaphoreType.DMA,
          pltpu.SemaphoreType.DMA,
      ],
  )
  def edge_pass(x_hbm, src_hbm, dst_hbm, outp_hbm, outd_hbm,
                sidx, didx, rowsa, rowsb, ones, zdeg, acc, deg, gsa, gsb):
    c = lax.axis_index("c")
    s = lax.axis_index("s")
    wid = c * NS + s

    z16 = jnp.zeros((16,), _f32)
    o16 = jnp.ones((16,), _f32)

    def zrow(i, _):
      for j in range(D // 16):
        rowsa[i, pl.ds(j * 16, 16)] = z16
      return 0
    lax.fori_loop(0, 128, zrow, 0)

    def zdg(i, _):
      zdeg[pl.ds(i * 16, 16)] = z16
      return 0
    lax.fori_loop(0, RPT // 16, zdg, 0)
    for j in range(128 // 16):
      ones[pl.ds(j * 16, 16)] = o16

    base = s * RPT
    for k in range(RPT // 128):
      pltpu.sync_copy(rowsa, acc.at[pl.ds(base + k * 128, 128)])
    pltpu.sync_copy(zdeg, deg.at[pl.ds(base, RPT)])
    plsc.subcore_barrier()

    pltpu.sync_copy(src_hbm.at[pl.ds(wid * CPT, CPT)], sidx)
    pltpu.sync_copy(dst_hbm.at[pl.ds(wid * CPT, CPT)], didx)

    def eloop(jj, _):
      j0 = 2 * jj
      j1 = 2 * jj + 1
      ga = pltpu.make_async_copy(x_hbm.at[sidx.at[j0]], rowsa, gsa)
      ga.start()
      gb = pltpu.make_async_copy(x_hbm.at[sidx.at[j1]], rowsb, gsb)
      gb.start()
      ga.wait()
      pltpu.sync_copy(rowsa, acc.at[didx.at[j0]], add=True)
      pltpu.sync_copy(ones, deg.at[didx.at[j0]], add=True)
      gb.wait()
      pltpu.sync_copy(rowsb, acc.at[didx.at[j1]], add=True)
      pltpu.sync_copy(ones, deg.at[didx.at[j1]], add=True)
      return 0
    lax.fori_loop(0, CPT // 2, eloop, 0)

    plsc.subcore_barrier()

    pltpu.sync_copy(acc.at[pl.ds(base, RPT)],
                    outp_hbm.at[pl.ds(c * NP + base, RPT)])
    pltpu.sync_copy(deg.at[pl.ds(base, RPT)],
                    outd_hbm.at[pl.ds(c * NP + base, RPT)])

  return edge_pass


def _make_dense(NP, D, H, BM, relu):
  NB = NP // BM

  def body(p0_ref, p1_ref, d0_ref, d1_ref, x_ref, wl_ref, wr_ref, b_ref,
           o_ref):
    d = d0_ref[0, 0, :] + d1_ref[0, 0, :]
    inv = 1.0 / jnp.maximum(d, 1.0)
    mean = (p0_ref[...] + p1_ref[...]) * inv[:, None]
    z = (jnp.dot(mean, wl_ref[...], preferred_element_type=_f32)
         + jnp.dot(x_ref[...], wr_ref[...], preferred_element_type=_f32)
         + b_ref[...])
    if relu:
      z = jnp.maximum(z, 0.0)
    o_ref[...] = z

  return pl.pallas_call(
      body,
      grid=(NB,),
      in_specs=[
          pl.BlockSpec((BM, D), lambda i: (i, 0)),
          pl.BlockSpec((BM, D), lambda i: (i + NB, 0)),
          pl.BlockSpec((1, 1, BM), lambda i: (i, 0, 0)),
          pl.BlockSpec((1, 1, BM), lambda i: (i + NB, 0, 0)),
          pl.BlockSpec((BM, D), lambda i: (i, 0)),
          pl.BlockSpec((D, H), lambda i: (0, 0)),
          pl.BlockSpec((D, H), lambda i: (0, 0)),
          pl.BlockSpec((1, H), lambda i: (0, 0)),
      ],
      out_specs=pl.BlockSpec((BM, H), lambda i: (i, 0)),
      out_shape=jax.ShapeDtypeStruct((NP, H), _f32),
  )


def _make_decode(NP, TE):
  EPT = TE // NW
  mesh = plsc.VectorSubcoreMesh(core_axis_name="c", subcore_axis_name="s")

  @functools.partial(
      pl.kernel,
      out_type=jax.ShapeDtypeStruct((TE,), _f32),
      mesh=mesh,
      scratch_types=[
          pltpu.VMEM((NP,), _f32),
          pltpu.VMEM((NP,), _f32),
          pltpu.VMEM((TE // NW,), jnp.int32),
          pltpu.VMEM((TE // NW,), jnp.int32),
          pltpu.VMEM((TE // NW,), _f32),
      ],
  )
  def decode(a_hbm, b_hbm, s_hbm, d_hbm, out_hbm, av, bv, si, di, ov):
    c = lax.axis_index("c")
    s = lax.axis_index("s")
    wid = c * NS + s
    base = wid * EPT
    pltpu.sync_copy(a_hbm, av)
    pltpu.sync_copy(b_hbm, bv)
    pltpu.sync_copy(s_hbm.at[pl.ds(base, EPT)], si)
    pltpu.sync_copy(d_hbm.at[pl.ds(base, EPT)], di)

    def loop(i, _):
      o = i * 16
      iv = si[pl.ds(o, 16)]
      jv = di[pl.ds(o, 16)]
      ov[pl.ds(o, 16)] = (plsc.load_gather(av, [iv])
                          + plsc.load_gather(bv, [jv]))
      return 0
    lax.fori_loop(0, EPT // 16, loop, 0)

    pltpu.sync_copy(ov, out_hbm.at[pl.ds(base, EPT)])

  return decode


def _pad_to(v, m):
  return ((v + m - 1) // m) * m


def kernel(x, edge_index, edge_weight, pos_edge_index, neg_edge_index,
           W1l, b1l, W1r, W2l, b2l, W2r, Wp, bp):
  N, D = x.shape
  H = W1l.shape[0]
  E = edge_index.shape[1]
  PE = pos_edge_index.shape[1]
  NE = neg_edge_index.shape[1]

  BM = 512
  NP = _pad_to(N, max(BM, NS * 128))
  EP = _pad_to(E, NW * 256)
  CPT = EP // (128 * NW)

  xp = jnp.zeros((NP, D), _f32).at[:N].set(x)

  npad = EP - E
  pad_src = (jnp.arange(npad, dtype=jnp.int32) * 97) % N
  pad_dst = N + (jnp.arange(npad, dtype=jnp.int32) % (NP - N))
  src = jnp.concatenate([edge_index[0], pad_src]).reshape(EP // 128, 128)
  dst = jnp.concatenate([edge_index[1], pad_dst]).reshape(EP // 128, 128)

  W1lT = W1l.T
  W1rT = W1r.T
  b1 = b1l.reshape(1, H)

  wa = Wp[0, :H]
  wb = Wp[0, H:]
  Ul = jnp.zeros((H, H), _f32).at[:, 0].set(W2l.T @ wa).at[:, 1].set(W2l.T @ wb)
  Ur = jnp.zeros((H, H), _f32).at[:, 0].set(W2r.T @ wa).at[:, 1].set(W2r.T @ wb)
  cvec = (jnp.zeros((1, H), _f32)
          .at[0, 0].set(b2l @ wa + bp[0])
          .at[0, 1].set(b2l @ wb))

  PP = _pad_to(PE, NW * 16)
  NPD = _pad_to(NE, NW * 16)
  TE = PP + NPD
  qsrc = jnp.zeros((TE,), jnp.int32)
  qsrc = qsrc.at[:PE].set(pos_edge_index[0]).at[PP:PP + NE].set(neg_edge_index[0])
  qdst = jnp.zeros((TE,), jnp.int32)
  qdst = qdst.at[:PE].set(pos_edge_index[1]).at[PP:PP + NE].set(neg_edge_index[1])

  edge_pass = _make_edge_pass(NP, D, CPT)
  dense1 = _make_dense(NP, D, H, BM, relu=True)
  dense2 = _make_dense(NP, H, H, BM, relu=False)
  decode = _make_decode(NP, TE)

  p, dg = edge_pass(xp, src, dst)
  d3 = dg.reshape(NC * NP // BM, 1, BM)
  z1 = dense1(p, p, d3, d3, xp, W1lT, W1rT, b1)

  p2, dg2 = edge_pass(z1, src, dst)
  d32 = dg2.reshape(NC * NP // BM, 1, BM)
  ab = dense2(p2, p2, d32, d32, z1, Ul, Ur, cvec)
  a = ab[:, 0]
  b = ab[:, 1]

  dec = decode(a, b, qsrc, qdst)
  pos = dec[:PE]
  neg = dec[PP:PP + NE]
  return (pos, neg)


# speedup vs baseline: 10.5735x; 10.5735x over previous
import functools

import jax
import jax.numpy as jnp
from jax import lax
from jax.experimental import pallas as pl
from jax.experimental.pallas import tpu as pltpu
from jax.experimental.pallas import tpu_sc as plsc

NC = 2
NS = 16
NW = NC * NS

_f32 = jnp.float32


def _make_edge_pass(NP, D, CPT, G=16):
  RPT = NP // NS
  mesh = plsc.VectorSubcoreMesh(core_axis_name="c", subcore_axis_name="s",
                                num_cores=NC, num_subcores=NS)

  @functools.partial(
      pl.kernel,
      out_type=(
          jax.ShapeDtypeStruct((NC * NP, D), _f32),
          jax.ShapeDtypeStruct((NC * NP,), _f32),
      ),
      mesh=mesh,
      scratch_types=[
          pltpu.VMEM((G, 128), jnp.int32),
          pltpu.VMEM((G, 128), jnp.int32),
          pltpu.VMEM((128, D), _f32),
          pltpu.VMEM((128, D), _f32),
          pltpu.VMEM((128,), _f32),
          pltpu.VMEM((NP // NS,), _f32),
          pltpu.VMEM_SHARED((NP, D), _f32),
          pltpu.VMEM_SHARED((NP,), _f32),
          pltpu.SemaphoreType.DMA,
          pltpu.SemaphoreType.DMA,
      ],
  )
  def edge_pass(x_hbm, src_hbm, dst_hbm, outp_hbm, outd_hbm,
                sidx, didx, rowsa, rowsb, ones, zdeg, acc, deg, gsa, gsb):
    c = lax.axis_index("c")
    s = lax.axis_index("s")
    wid = c * NS + s

    z16 = jnp.zeros((16,), _f32)
    o16 = jnp.ones((16,), _f32)

    def zrow(i, _):
      for j in range(D // 16):
        rowsa[i, pl.ds(j * 16, 16)] = z16
      return 0
    lax.fori_loop(0, 128, zrow, 0)

    def zdg(i, _):
      zdeg[pl.ds(i * 16, 16)] = z16
      return 0
    lax.fori_loop(0, RPT // 16, zdg, 0)
    for j in range(128 // 16):
      ones[pl.ds(j * 16, 16)] = o16

    base = s * RPT
    for k in range(RPT // 128):
      pltpu.sync_copy(rowsa, acc.at[pl.ds(base + k * 128, 128)])
    pltpu.sync_copy(zdeg, deg.at[pl.ds(base, RPT)])
    plsc.subcore_barrier()

    def gloop(g, _):
      gb_ = wid * CPT + g * G
      pltpu.sync_copy(src_hbm.at[pl.ds(gb_, G)], sidx)
      pltpu.sync_copy(dst_hbm.at[pl.ds(gb_, G)], didx)

      def eloop(jj, _):
        j0 = 2 * jj
        j1 = 2 * jj + 1
        ga = pltpu.make_async_copy(x_hbm.at[sidx.at[j0]], rowsa, gsa)
        ga.start()
        gb = pltpu.make_async_copy(x_hbm.at[sidx.at[j1]], rowsb, gsb)
        gb.start()
        ga.wait()
        pltpu.sync_copy(rowsa, acc.at[didx.at[j0]], add=True)
        pltpu.sync_copy(ones, deg.at[didx.at[j0]], add=True)
        gb.wait()
        pltpu.sync_copy(rowsb, acc.at[didx.at[j1]], add=True)
        pltpu.sync_copy(ones, deg.at[didx.at[j1]], add=True)
        return 0
      lax.fori_loop(0, G // 2, eloop, 0)
      return 0
    lax.fori_loop(0, CPT // G, gloop, 0)

    plsc.subcore_barrier()

    pltpu.sync_copy(acc.at[pl.ds(base, RPT)],
                    outp_hbm.at[pl.ds(c * NP + base, RPT)])
    pltpu.sync_copy(deg.at[pl.ds(base, RPT)],
                    outd_hbm.at[pl.ds(c * NP + base, RPT)])

  return edge_pass


def _make_dense(NP, D, H, BM, relu):
  NB = NP // BM

  def body(p0_ref, p1_ref, d0_ref, d1_ref, x_ref, wl_ref, wr_ref, b_ref,
           o_ref):
    d = d0_ref[0, 0, :] + d1_ref[0, 0, :]
    inv = 1.0 / jnp.maximum(d, 1.0)
    mean = (p0_ref[...] + p1_ref[...]) * inv[:, None]
    z = (jnp.dot(mean, wl_ref[...], preferred_element_type=_f32)
         + jnp.dot(x_ref[...], wr_ref[...], preferred_element_type=_f32)
         + b_ref[...])
    if relu:
      z = jnp.maximum(z, 0.0)
    o_ref[...] = z

  return pl.pallas_call(
      body,
      grid=(NB,),
      in_specs=[
          pl.BlockSpec((BM, D), lambda i: (i, 0)),
          pl.BlockSpec((BM, D), lambda i: (i + NB, 0)),
          pl.BlockSpec((1, 1, BM), lambda i: (i, 0, 0)),
          pl.BlockSpec((1, 1, BM), lambda i: (i + NB, 0, 0)),
          pl.BlockSpec((BM, D), lambda i: (i, 0)),
          pl.BlockSpec((D, H), lambda i: (0, 0)),
          pl.BlockSpec((D, H), lambda i: (0, 0)),
          pl.BlockSpec((1, H), lambda i: (0, 0)),
      ],
      out_specs=pl.BlockSpec((BM, H), lambda i: (i, 0)),
      out_shape=jax.ShapeDtypeStruct((NP, H), _f32),
  )


def _make_decode(NP, TE):
  EPT = TE // NW
  CQ = EPT // 128
  mesh = plsc.VectorSubcoreMesh(core_axis_name="c", subcore_axis_name="s",
                                num_cores=NC, num_subcores=NS)

  @functools.partial(
      pl.kernel,
      out_type=jax.ShapeDtypeStruct((TE,), _f32),
      mesh=mesh,
      scratch_types=[
          pltpu.VMEM((CQ, 128), jnp.int32),
          pltpu.VMEM((CQ, 128), jnp.int32),
          pltpu.VMEM((128,), _f32),
          pltpu.VMEM((128,), _f32),
          pltpu.VMEM((EPT,), _f32),
          pltpu.SemaphoreType.DMA,
          pltpu.SemaphoreType.DMA,
      ],
  )
  def decode(a_hbm, b_hbm, s_hbm, d_hbm, out_hbm, si, di, bufa, bufb, ov,
             sma, smb):
    c = lax.axis_index("c")
    s = lax.axis_index("s")
    wid = c * NS + s
    pltpu.sync_copy(s_hbm.at[pl.ds(wid * CQ, CQ)], si)
    pltpu.sync_copy(d_hbm.at[pl.ds(wid * CQ, CQ)], di)

    def loop(j, _):
      ga = pltpu.make_async_copy(a_hbm.at[si.at[j]], bufa, sma)
      ga.start()
      gb = pltpu.make_async_copy(b_hbm.at[di.at[j]], bufb, smb)
      gb.start()
      ga.wait()
      gb.wait()
      for k in range(128 // 16):
        o = k * 16
        ov[pl.ds(j * 128 + o, 16)] = (bufa[pl.ds(o, 16)]
                                      + bufb[pl.ds(o, 16)])
      return 0
    lax.fori_loop(0, CQ, loop, 0)

    pltpu.sync_copy(ov, out_hbm.at[pl.ds(wid * EPT, EPT)])

  return decode


def _pad_to(v, m):
  return ((v + m - 1) // m) * m


def kernel(x, edge_index, edge_weight, pos_edge_index, neg_edge_index,
           W1l, b1l, W1r, W2l, b2l, W2r, Wp, bp):
  N, D = x.shape
  H = W1l.shape[0]
  E = edge_index.shape[1]
  PE = pos_edge_index.shape[1]
  NE = neg_edge_index.shape[1]

  BM = 512
  NP = _pad_to(N, max(BM, NS * 128))
  EP = _pad_to(E, NW * 256)
  CPT = EP // (128 * NW)

  xp = jnp.zeros((NP, D), _f32).at[:N].set(x)

  npad = EP - E
  pad_src = (jnp.arange(npad, dtype=jnp.int32) * 97) % N
  pad_dst = N + (jnp.arange(npad, dtype=jnp.int32) % (NP - N))
  src = jnp.concatenate([edge_index[0], pad_src]).reshape(EP // 128, 128)
  dst = jnp.concatenate([edge_index[1], pad_dst]).reshape(EP // 128, 128)

  W1lT = W1l.T
  W1rT = W1r.T
  b1 = b1l.reshape(1, H)

  wa = Wp[0, :H]
  wb = Wp[0, H:]
  Ul = jnp.zeros((H, H), _f32).at[:, 0].set(W2l.T @ wa).at[:, 1].set(W2l.T @ wb)
  Ur = jnp.zeros((H, H), _f32).at[:, 0].set(W2r.T @ wa).at[:, 1].set(W2r.T @ wb)
  cvec = (jnp.zeros((1, H), _f32)
          .at[0, 0].set(b2l @ wa + bp[0])
          .at[0, 1].set(b2l @ wb))

  PP = _pad_to(PE, 128)
  TE = _pad_to(PP + NE, NW * 8 * 128)
  fill = (jnp.arange(TE, dtype=jnp.int32) * 89) % N
  qsrc = fill.at[:PE].set(pos_edge_index[0]).at[PP:PP + NE].set(neg_edge_index[0])
  qdst = fill.at[:PE].set(pos_edge_index[1]).at[PP:PP + NE].set(neg_edge_index[1])
  qsrc = qsrc.reshape(TE // 128, 128)
  qdst = qdst.reshape(TE // 128, 128)

  edge_pass = _make_edge_pass(NP, D, CPT)
  dense1 = _make_dense(NP, D, H, BM, relu=True)
  dense2 = _make_dense(NP, H, H, BM, relu=False)
  decode = _make_decode(NP, TE)

  p, dg = edge_pass(xp, src, dst)
  d3 = dg.reshape(NC * NP // BM, 1, BM)
  z1 = dense1(p, p, d3, d3, xp, W1lT, W1rT, b1)

  p2, dg2 = edge_pass(z1, src, dst)
  d32 = dg2.reshape(NC * NP // BM, 1, BM)
  ab = dense2(p2, p2, d32, d32, z1, Ul, Ur, cvec)
  a = ab[:, 0]
  b = ab[:, 1]

  dec = decode(a, b, qsrc, qdst)
  pos = dec[:PE]
  neg = dec[PP:PP + NE]
  return (pos, neg)


# speedup vs baseline: 11.9659x; 1.1317x over previous
import functools

import jax
import jax.numpy as jnp
from jax import lax
from jax.experimental import pallas as pl
from jax.experimental.pallas import tpu as pltpu
from jax.experimental.pallas import tpu_sc as plsc

NC = 2
NS = 16
NW = NC * NS

_f32 = jnp.float32


def _make_edge_pass(NP, D, CPT, G=16, with_deg=True):
  RPT = NP // NS
  mesh = plsc.VectorSubcoreMesh(core_axis_name="c", subcore_axis_name="s",
                                num_cores=NC, num_subcores=NS)

  out_type = [jax.ShapeDtypeStruct((NC * NP, D), _f32)]
  if with_deg:
    out_type.append(jax.ShapeDtypeStruct((NC * NP,), _f32))

  @functools.partial(
      pl.kernel,
      out_type=tuple(out_type),
      mesh=mesh,
      scratch_types=[
          pltpu.VMEM((G, 128), jnp.int32),
          pltpu.VMEM((G, 128), jnp.int32),
          pltpu.VMEM((128, D), _f32),
          pltpu.VMEM((128, D), _f32),
          pltpu.VMEM((128,), _f32),
          pltpu.VMEM_SHARED((NP, D), _f32),
          pltpu.VMEM_SHARED((NP,), _f32),
          pltpu.SemaphoreType.DMA,
          pltpu.SemaphoreType.DMA,
          pltpu.SemaphoreType.DMA,
          pltpu.SemaphoreType.DMA,
      ],
  )
  def edge_pass(x_hbm, src_hbm, dst_hbm, z2d_hbm, z1d_hbm, on_hbm, outp_hbm,
                *refs):
    if with_deg:
      outd_hbm = refs[0]
      refs = refs[1:]
    sidx, didx, rowsa, rowsb, ones, acc, deg, gsa, gsb, ssa, ssb = refs
    c = lax.axis_index("c")
    s = lax.axis_index("s")
    wid = c * NS + s

    base = s * RPT
    pltpu.sync_copy(z2d_hbm, acc.at[pl.ds(base, RPT)])
    if with_deg:
      pltpu.sync_copy(z1d_hbm, deg.at[pl.ds(base, RPT)])
      pltpu.sync_copy(on_hbm, ones)
    plsc.subcore_barrier()

    def gloop(g, _):
      gb_ = wid * CPT + g * G
      pltpu.sync_copy(src_hbm.at[pl.ds(gb_, G)], sidx)
      pltpu.sync_copy(dst_hbm.at[pl.ds(gb_, G)], didx)
      pltpu.make_async_copy(x_hbm.at[sidx.at[0]], rowsa, gsa).start()
      pltpu.make_async_copy(x_hbm.at[sidx.at[1]], rowsb, gsb).start()

      def eloop(jj, _):
        j0 = 2 * jj
        j1 = j0 + 1
        pltpu.make_async_copy(x_hbm.at[sidx.at[j0]], rowsa, gsa).wait()
        sca = pltpu.make_async_copy(rowsa, acc.at[didx.at[j0]], ssa)
        sca.start(add=True)
        if with_deg:
          pltpu.sync_copy(ones, deg.at[didx.at[j0]], add=True)
        pltpu.make_async_copy(x_hbm.at[sidx.at[j1]], rowsb, gsb).wait()
        scb = pltpu.make_async_copy(rowsb, acc.at[didx.at[j1]], ssb)
        scb.start(add=True)
        if with_deg:
          pltpu.sync_copy(ones, deg.at[didx.at[j1]], add=True)
        sca.wait()

        @pl.when(j0 + 2 < G)
        def _():
          pltpu.make_async_copy(x_hbm.at[sidx.at[j0 + 2]], rowsa, gsa).start()
        scb.wait()

        @pl.when(j1 + 2 < G)
        def _():
          pltpu.make_async_copy(x_hbm.at[sidx.at[j1 + 2]], rowsb, gsb).start()
        return 0
      lax.fori_loop(0, G // 2, eloop, 0)
      return 0
    lax.fori_loop(0, CPT // G, gloop, 0)

    plsc.subcore_barrier()

    pltpu.sync_copy(acc.at[pl.ds(base, RPT)],
                    outp_hbm.at[pl.ds(c * NP + base, RPT)])
    if with_deg:
      pltpu.sync_copy(deg.at[pl.ds(base, RPT)],
                      outd_hbm.at[pl.ds(c * NP + base, RPT)])

  return edge_pass


def _make_dense(NP, D, H, BM, relu):
  NB = NP // BM

  def body(p0_ref, p1_ref, d0_ref, d1_ref, x_ref, wl_ref, wr_ref, b_ref,
           o_ref):
    d = d0_ref[0, 0, :] + d1_ref[0, 0, :]
    inv = 1.0 / jnp.maximum(d, 1.0)
    mean = (p0_ref[...] + p1_ref[...]) * inv[:, None]
    z = (jnp.dot(mean, wl_ref[...], preferred_element_type=_f32)
         + jnp.dot(x_ref[...], wr_ref[...], preferred_element_type=_f32)
         + b_ref[...])
    if relu:
      z = jnp.maximum(z, 0.0)
    o_ref[...] = z

  return pl.pallas_call(
      body,
      grid=(NB,),
      in_specs=[
          pl.BlockSpec((BM, D), lambda i: (i, 0)),
          pl.BlockSpec((BM, D), lambda i: (i + NB, 0)),
          pl.BlockSpec((1, 1, BM), lambda i: (i, 0, 0)),
          pl.BlockSpec((1, 1, BM), lambda i: (i + NB, 0, 0)),
          pl.BlockSpec((BM, D), lambda i: (i, 0)),
          pl.BlockSpec((D, H), lambda i: (0, 0)),
          pl.BlockSpec((D, H), lambda i: (0, 0)),
          pl.BlockSpec((1, H), lambda i: (0, 0)),
      ],
      out_specs=pl.BlockSpec((BM, H), lambda i: (i, 0)),
      out_shape=jax.ShapeDtypeStruct((NP, H), _f32),
  )


def _make_decode(NP, TE, GQ=8):
  EPT = TE // NW
  CQ = EPT // 128
  mesh = plsc.VectorSubcoreMesh(core_axis_name="c", subcore_axis_name="s",
                                num_cores=NC, num_subcores=NS)

  @functools.partial(
      pl.kernel,
      out_type=jax.ShapeDtypeStruct((TE,), _f32),
      mesh=mesh,
      scratch_types=[
          pltpu.VMEM((CQ, 128), jnp.int32),
          pltpu.VMEM((CQ, 128), jnp.int32),
          pltpu.VMEM((GQ * 128,), _f32),
          pltpu.VMEM((GQ * 128,), _f32),
          pltpu.VMEM((EPT,), _f32),
          pltpu.VMEM_SHARED((NP,), _f32),
          pltpu.VMEM_SHARED((NP,), _f32),
          pltpu.SemaphoreType.DMA,
          pltpu.SemaphoreType.DMA,
      ],
  )
  def decode(a_hbm, b_hbm, s_hbm, d_hbm, out_hbm, si, di, bufa, bufb, ov,
             ash, bsh, sma, smb):
    c = lax.axis_index("c")
    s = lax.axis_index("s")
    wid = c * NS + s
    @pl.when(s == 0)
    def _():
      pltpu.sync_copy(a_hbm, ash)
      pltpu.sync_copy(b_hbm, bsh)
    pltpu.sync_copy(s_hbm.at[pl.ds(wid * CQ, CQ)], si)
    pltpu.sync_copy(d_hbm.at[pl.ds(wid * CQ, CQ)], di)
    plsc.subcore_barrier()

    def loop(g, _):
      j0 = g * GQ
      for k in range(GQ):
        pltpu.make_async_copy(ash.at[si.at[j0 + k]],
                              bufa.at[pl.ds(k * 128, 128)], sma).start()
        pltpu.make_async_copy(bsh.at[di.at[j0 + k]],
                              bufb.at[pl.ds(k * 128, 128)], smb).start()
      for k in range(GQ):
        pltpu.make_async_copy(ash.at[si.at[j0 + k]],
                              bufa.at[pl.ds(k * 128, 128)], sma).wait()
        pltpu.make_async_copy(bsh.at[di.at[j0 + k]],
                              bufb.at[pl.ds(k * 128, 128)], smb).wait()
      for k in range(GQ * 128 // 16):
        o = k * 16
        ov[pl.ds(j0 * 128 + o, 16)] = (bufa[pl.ds(o, 16)]
                                       + bufb[pl.ds(o, 16)])
      return 0
    lax.fori_loop(0, CQ // GQ, loop, 0)

    pltpu.sync_copy(ov, out_hbm.at[pl.ds(wid * EPT, EPT)])

  return decode


def _pad_to(v, m):
  return ((v + m - 1) // m) * m


def kernel(x, edge_index, edge_weight, pos_edge_index, neg_edge_index,
           W1l, b1l, W1r, W2l, b2l, W2r, Wp, bp):
  N, D = x.shape
  H = W1l.shape[0]
  E = edge_index.shape[1]
  PE = pos_edge_index.shape[1]
  NE = neg_edge_index.shape[1]

  BM = 512
  NP = _pad_to(N, max(BM, NS * 128))
  EP = _pad_to(E, NW * 256)
  CPT = EP // (128 * NW)

  xp = jnp.zeros((NP, D), _f32).at[:N].set(x)

  npad = EP - E
  pad_src = (jnp.arange(npad, dtype=jnp.int32) * 97) % N
  pad_dst = N + (jnp.arange(npad, dtype=jnp.int32) % (NP - N))
  src = jnp.concatenate([edge_index[0], pad_src]).reshape(EP // 128, 128)
  dst = jnp.concatenate([edge_index[1], pad_dst]).reshape(EP // 128, 128)

  W1lT = W1l.T
  W1rT = W1r.T
  b1 = b1l.reshape(1, H)

  wa = Wp[0, :H]
  wb = Wp[0, H:]
  Ul = jnp.zeros((H, H), _f32).at[:, 0].set(W2l.T @ wa).at[:, 1].set(W2l.T @ wb)
  Ur = jnp.zeros((H, H), _f32).at[:, 0].set(W2r.T @ wa).at[:, 1].set(W2r.T @ wb)
  cvec = (jnp.zeros((1, H), _f32)
          .at[0, 0].set(b2l @ wa + bp[0])
          .at[0, 1].set(b2l @ wb))

  PP = _pad_to(PE, 128)
  TE = _pad_to(PP + NE, NW * 8 * 128)
  fill = (jnp.arange(TE, dtype=jnp.int32) * 89) % N
  qsrc = fill.at[:PE].set(pos_edge_index[0]).at[PP:PP + NE].set(neg_edge_index[0])
  qdst = fill.at[:PE].set(pos_edge_index[1]).at[PP:PP + NE].set(neg_edge_index[1])
  qsrc = qsrc.reshape(TE // 128, 128)
  qdst = qdst.reshape(TE // 128, 128)

  zr2 = jnp.zeros((NP // NS, D), _f32)
  zr1 = jnp.zeros((NP // NS,), _f32)
  on1 = jnp.ones((128,), _f32)

  edge_pass = _make_edge_pass(NP, D, CPT, with_deg=True)
  edge_pass2 = _make_edge_pass(NP, D, CPT, with_deg=False)
  dense1 = _make_dense(NP, D, H, BM, relu=True)
  dense2 = _make_dense(NP, H, H, BM, relu=False)
  decode = _make_decode(NP, TE)

  p, dg = edge_pass(xp, src, dst, zr2, zr1, on1)
  d3 = dg.reshape(NC * NP // BM, 1, BM)
  z1 = dense1(p, p, d3, d3, xp, W1lT, W1rT, b1)

  (p2,) = edge_pass2(z1, src, dst, zr2, zr1, on1)
  ab = dense2(p2, p2, d3, d3, z1, Ul, Ur, cvec)
  a = ab[:, 0]
  b = ab[:, 1]

  dec = decode(a, b, qsrc, qdst)
  pos = dec[:PE]
  neg = dec[PP:PP + NE]
  return (pos, neg)


# speedup vs baseline: 18.1800x; 1.5193x over previous
import functools

import jax
import jax.numpy as jnp
from jax import lax
from jax.experimental import pallas as pl
from jax.experimental.pallas import tpu as pltpu
from jax.experimental.pallas import tpu_sc as plsc

NC = 2
NS = 16
NW = NC * NS

_f32 = jnp.float32


def _make_edge_pass(NP, D, CPT, G=16, with_deg=True):
  RPT = NP // NS
  mesh = plsc.VectorSubcoreMesh(core_axis_name="c", subcore_axis_name="s",
                                num_cores=NC, num_subcores=NS)

  out_type = [jax.ShapeDtypeStruct((NC * NP, D), _f32)]
  if with_deg:
    out_type.append(jax.ShapeDtypeStruct((NC * NP,), _f32))

  @functools.partial(
      pl.kernel,
      out_type=tuple(out_type),
      mesh=mesh,
      scratch_types=[
          pltpu.VMEM((G, 128), jnp.int32),
          pltpu.VMEM((G, 128), jnp.int32),
          pltpu.VMEM((128, D), _f32),
          pltpu.VMEM((128, D), _f32),
          pltpu.VMEM((128,), _f32),
          pltpu.VMEM_SHARED((NP, D), _f32),
          pltpu.VMEM_SHARED((NP,), _f32),
          pltpu.SemaphoreType.DMA,
          pltpu.SemaphoreType.DMA,
          pltpu.SemaphoreType.DMA,
          pltpu.SemaphoreType.DMA,
      ],
  )
  def edge_pass(x_hbm, src_hbm, dst_hbm, z2d_hbm, z1d_hbm, on_hbm, outp_hbm,
                *refs):
    if with_deg:
      outd_hbm = refs[0]
      refs = refs[1:]
    sidx, didx, rowsa, rowsb, ones, acc, deg, gsa, gsb, ssa, ssb = refs
    c = lax.axis_index("c")
    s = lax.axis_index("s")
    wid = c * NS + s

    base = s * RPT
    pltpu.sync_copy(z2d_hbm, acc.at[pl.ds(base, RPT)])
    if with_deg:
      pltpu.sync_copy(z1d_hbm, deg.at[pl.ds(base, RPT)])
      pltpu.sync_copy(on_hbm, ones)
    plsc.subcore_barrier()

    def gloop(g, _):
      gb_ = wid * CPT + g * G
      pltpu.sync_copy(src_hbm.at[pl.ds(gb_, G)], sidx)
      pltpu.sync_copy(dst_hbm.at[pl.ds(gb_, G)], didx)
      pltpu.make_async_copy(x_hbm.at[sidx.at[0]], rowsa, gsa).start()
      pltpu.make_async_copy(x_hbm.at[sidx.at[1]], rowsb, gsb).start()

      def eloop(jj, _):
        j0 = 2 * jj
        j1 = j0 + 1
        pltpu.make_async_copy(x_hbm.at[sidx.at[j0]], rowsa, gsa).wait()
        sca = pltpu.make_async_copy(rowsa, acc.at[didx.at[j0]], ssa)
        sca.start(add=True)
        if with_deg:
          pltpu.sync_copy(ones, deg.at[didx.at[j0]], add=True)
        pltpu.make_async_copy(x_hbm.at[sidx.at[j1]], rowsb, gsb).wait()
        scb = pltpu.make_async_copy(rowsb, acc.at[didx.at[j1]], ssb)
        scb.start(add=True)
        if with_deg:
          pltpu.sync_copy(ones, deg.at[didx.at[j1]], add=True)
        sca.wait()

        @pl.when(j0 + 2 < G)
        def _():
          pltpu.make_async_copy(x_hbm.at[sidx.at[j0 + 2]], rowsa, gsa).start()
        scb.wait()

        @pl.when(j1 + 2 < G)
        def _():
          pltpu.make_async_copy(x_hbm.at[sidx.at[j1 + 2]], rowsb, gsb).start()
        return 0
      lax.fori_loop(0, G // 2, eloop, 0)
      return 0
    lax.fori_loop(0, CPT // G, gloop, 0)

    plsc.subcore_barrier()

    pltpu.sync_copy(acc.at[pl.ds(base, RPT)],
                    outp_hbm.at[pl.ds(c * NP + base, RPT)])
    if with_deg:
      pltpu.sync_copy(deg.at[pl.ds(base, RPT)],
                      outd_hbm.at[pl.ds(c * NP + base, RPT)])

  return edge_pass


def _make_dense1(NP, D, H, BM):
  NB = NP // BM

  def body(p0_ref, p1_ref, d0_ref, d1_ref, x_ref, wl_ref, wr_ref, b_ref,
           u_ref, o_ref, pr_ref):
    d = d0_ref[0, 0, :] + d1_ref[0, 0, :]
    inv = 1.0 / jnp.maximum(d, 1.0)
    mean = (p0_ref[...] + p1_ref[...]) * inv[:, None]
    z = (jnp.dot(mean, wl_ref[...], preferred_element_type=_f32)
         + jnp.dot(x_ref[...], wr_ref[...], preferred_element_type=_f32)
         + b_ref[...])
    z = jnp.maximum(z, 0.0)
    o_ref[...] = z
    pr_ref[...] = jax.lax.dot_general(
        u_ref[...], z, (((1,), (1,)), ((), ())),
        preferred_element_type=_f32)

  return pl.pallas_call(
      body,
      grid=(NB,),
      in_specs=[
          pl.BlockSpec((BM, D), lambda i: (i, 0)),
          pl.BlockSpec((BM, D), lambda i: (i + NB, 0)),
          pl.BlockSpec((1, 1, BM), lambda i: (i, 0, 0)),
          pl.BlockSpec((1, 1, BM), lambda i: (i + NB, 0, 0)),
          pl.BlockSpec((BM, D), lambda i: (i, 0)),
          pl.BlockSpec((D, H), lambda i: (0, 0)),
          pl.BlockSpec((D, H), lambda i: (0, 0)),
          pl.BlockSpec((1, H), lambda i: (0, 0)),
          pl.BlockSpec((8, H), lambda i: (0, 0)),
      ],
      out_specs=[
          pl.BlockSpec((BM, H), lambda i: (i, 0)),
          pl.BlockSpec((8, BM), lambda i: (0, i)),
      ],
      out_shape=[
          jax.ShapeDtypeStruct((NP, H), _f32),
          jax.ShapeDtypeStruct((8, NP), _f32),
      ],
  )


def _make_edge_narrow(NP, CPT, GQ=8):
  RPT = NP // NS
  mesh = plsc.VectorSubcoreMesh(core_axis_name="c", subcore_axis_name="s",
                                num_cores=NC, num_subcores=NS)

  @functools.partial(
      pl.kernel,
      out_type=(
          jax.ShapeDtypeStruct((NC * NP,), _f32),
          jax.ShapeDtypeStruct((NC * NP,), _f32),
      ),
      mesh=mesh,
      scratch_types=[
          pltpu.VMEM((CPT, 128), jnp.int32),
          pltpu.VMEM((CPT, 128), jnp.int32),
          pltpu.VMEM((GQ * 128,), _f32),
          pltpu.VMEM((GQ * 128,), _f32),
          pltpu.VMEM_SHARED((NP,), _f32),
          pltpu.VMEM_SHARED((NP,), _f32),
          pltpu.VMEM_SHARED((NP,), _f32),
          pltpu.VMEM_SHARED((NP,), _f32),
          pltpu.SemaphoreType.DMA,
          pltpu.SemaphoreType.DMA,
          pltpu.SemaphoreType.DMA,
          pltpu.SemaphoreType.DMA,
      ],
  )
  def narrow(t_hbm, src_hbm, dst_hbm, z1d_hbm, outa_hbm, outb_hbm,
             si, di, bufa, bufb, ash, bsh, acca, accb, sma, smb, swa, swb):
    c = lax.axis_index("c")
    s = lax.axis_index("s")
    wid = c * NS + s
    base = s * RPT

    @pl.when(s == 0)
    def _():
      pltpu.sync_copy(t_hbm.at[0], ash)
      pltpu.sync_copy(t_hbm.at[1], bsh)
    pltpu.sync_copy(z1d_hbm, acca.at[pl.ds(base, RPT)])
    pltpu.sync_copy(z1d_hbm, accb.at[pl.ds(base, RPT)])
    pltpu.sync_copy(src_hbm.at[pl.ds(wid * CPT, CPT)], si)
    pltpu.sync_copy(dst_hbm.at[pl.ds(wid * CPT, CPT)], di)
    plsc.subcore_barrier()

    def loop(g, _):
      j0 = g * GQ

      @pl.when(g > 0)
      def _():
        for k in range(GQ):
          jp = j0 - GQ + k
          pltpu.make_async_copy(bufa.at[pl.ds(k * 128, 128)],
                                acca.at[di.at[jp]], swa).wait()
          pltpu.make_async_copy(bufb.at[pl.ds(k * 128, 128)],
                                accb.at[di.at[jp]], swb).wait()
      for k in range(GQ):
        pltpu.make_async_copy(ash.at[si.at[j0 + k]],
                              bufa.at[pl.ds(k * 128, 128)], sma).start()
        pltpu.make_async_copy(bsh.at[si.at[j0 + k]],
                              bufb.at[pl.ds(k * 128, 128)], smb).start()
      for k in range(GQ):
        pltpu.make_async_copy(ash.at[si.at[j0 + k]],
                              bufa.at[pl.ds(k * 128, 128)], sma).wait()
        pltpu.make_async_copy(bsh.at[si.at[j0 + k]],
                              bufb.at[pl.ds(k * 128, 128)], smb).wait()
      for k in range(GQ):
        pltpu.make_async_copy(bufa.at[pl.ds(k * 128, 128)],
                              acca.at[di.at[j0 + k]], swa).start(add=True)
        pltpu.make_async_copy(bufb.at[pl.ds(k * 128, 128)],
                              accb.at[di.at[j0 + k]], swb).start(add=True)
      return 0
    lax.fori_loop(0, CPT // GQ, loop, 0)

    for k in range(GQ):
      jp = CPT - GQ + k
      pltpu.make_async_copy(bufa.at[pl.ds(k * 128, 128)],
                            acca.at[di.at[jp]], swa).wait()
      pltpu.make_async_copy(bufb.at[pl.ds(k * 128, 128)],
                            accb.at[di.at[jp]], swb).wait()
    plsc.subcore_barrier()

    pltpu.sync_copy(acca.at[pl.ds(base, RPT)],
                    outa_hbm.at[pl.ds(c * NP + base, RPT)])
    pltpu.sync_copy(accb.at[pl.ds(base, RPT)],
                    outb_hbm.at[pl.ds(c * NP + base, RPT)])

  return narrow


def _make_dense2(NP, BM):
  NB = NP // BM

  def body(pa0_ref, pa1_ref, pb0_ref, pb1_ref, d0_ref, d1_ref, pr_ref,
           c_ref, o_ref):
    d = d0_ref[0, 0, :] + d1_ref[0, 0, :]
    inv = 1.0 / jnp.maximum(d, 1.0)
    pr = pr_ref[...]
    carr = c_ref[...]
    aa = ((pa0_ref[0, 0, :] + pa1_ref[0, 0, :]) * inv + pr[2, :]
          + carr[0, 0:1])
    bb = ((pb0_ref[0, 0, :] + pb1_ref[0, 0, :]) * inv + pr[3, :]
          + carr[0, 1:2])
    o_ref[...] = jnp.concatenate(
        [aa[None, :], bb[None, :], jnp.zeros((6, aa.shape[0]), _f32)], axis=0)

  return pl.pallas_call(
      body,
      grid=(NB,),
      in_specs=[
          pl.BlockSpec((1, 1, BM), lambda i: (i, 0, 0)),
          pl.BlockSpec((1, 1, BM), lambda i: (i + NB, 0, 0)),
          pl.BlockSpec((1, 1, BM), lambda i: (i, 0, 0)),
          pl.BlockSpec((1, 1, BM), lambda i: (i + NB, 0, 0)),
          pl.BlockSpec((1, 1, BM), lambda i: (i, 0, 0)),
          pl.BlockSpec((1, 1, BM), lambda i: (i + NB, 0, 0)),
          pl.BlockSpec((8, BM), lambda i: (0, i)),
          pl.BlockSpec((1, 8), lambda i: (0, 0)),
      ],
      out_specs=pl.BlockSpec((8, BM), lambda i: (0, i)),
      out_shape=jax.ShapeDtypeStruct((8, NP), _f32),
  )


def _make_decode(NPF, TE, GQ=8):
  EPT = TE // NW
  CQ = EPT // 128
  mesh = plsc.VectorSubcoreMesh(core_axis_name="c", subcore_axis_name="s",
                                num_cores=NC, num_subcores=NS)

  @functools.partial(
      pl.kernel,
      out_type=jax.ShapeDtypeStruct((TE,), _f32),
      mesh=mesh,
      scratch_types=[
          pltpu.VMEM((CQ, 128), jnp.int32),
          pltpu.VMEM((CQ, 128), jnp.int32),
          pltpu.VMEM((GQ * 128,), _f32),
          pltpu.VMEM((GQ * 128,), _f32),
          pltpu.VMEM((EPT,), _f32),
          pltpu.VMEM_SHARED((NPF,), _f32),
          pltpu.SemaphoreType.DMA,
          pltpu.SemaphoreType.DMA,
      ],
  )
  def decode(t_hbm, s_hbm, d_hbm, out_hbm, si, di, bufa, bufb, ov,
             tsh, sma, smb):
    c = lax.axis_index("c")
    s = lax.axis_index("s")
    wid = c * NS + s
    @pl.when(s == 0)
    def _():
      pltpu.sync_copy(t_hbm, tsh)
    pltpu.sync_copy(s_hbm.at[pl.ds(wid * CQ, CQ)], si)
    pltpu.sync_copy(d_hbm.at[pl.ds(wid * CQ, CQ)], di)
    plsc.subcore_barrier()

    def loop(g, _):
      j0 = g * GQ
      for k in range(GQ):
        pltpu.make_async_copy(tsh.at[si.at[j0 + k]],
                              bufa.at[pl.ds(k * 128, 128)], sma).start()
        pltpu.make_async_copy(tsh.at[di.at[j0 + k]],
                              bufb.at[pl.ds(k * 128, 128)], smb).start()
      for k in range(GQ):
        pltpu.make_async_copy(tsh.at[si.at[j0 + k]],
                              bufa.at[pl.ds(k * 128, 128)], sma).wait()
        pltpu.make_async_copy(tsh.at[di.at[j0 + k]],
                              bufb.at[pl.ds(k * 128, 128)], smb).wait()
      for k in range(GQ * 128 // 16):
        o = k * 16
        ov[pl.ds(j0 * 128 + o, 16)] = (bufa[pl.ds(o, 16)]
                                       + bufb[pl.ds(o, 16)])
      return 0
    lax.fori_loop(0, CQ // GQ, loop, 0)

    pltpu.sync_copy(ov, out_hbm.at[pl.ds(wid * EPT, EPT)])

  return decode


def _pad_to(v, m):
  return ((v + m - 1) // m) * m


def kernel(x, edge_index, edge_weight, pos_edge_index, neg_edge_index,
           W1l, b1l, W1r, W2l, b2l, W2r, Wp, bp):
  N, D = x.shape
  H = W1l.shape[0]
  E = edge_index.shape[1]
  PE = pos_edge_index.shape[1]
  NE = neg_edge_index.shape[1]

  BM = 512
  NP = _pad_to(N, max(BM, NS * 128))
  EP = _pad_to(E, NW * 256)
  CPT = EP // (128 * NW)

  xp = jnp.zeros((NP, D), _f32).at[:N].set(x)

  npad = EP - E
  pad_src = (jnp.arange(npad, dtype=jnp.int32) * 97) % N
  pad_dst = N + (jnp.arange(npad, dtype=jnp.int32) % (NP - N))
  src = jnp.concatenate([edge_index[0], pad_src]).reshape(EP // 128, 128)
  dst = jnp.concatenate([edge_index[1], pad_dst]).reshape(EP // 128, 128)

  W1lT = W1l.T
  W1rT = W1r.T
  b1 = b1l.reshape(1, H)

  wa = Wp[0, :H]
  wb = Wp[0, H:]
  U8 = (jnp.zeros((8, H), _f32)
        .at[0].set(W2l.T @ wa).at[1].set(W2l.T @ wb)
        .at[2].set(W2r.T @ wa).at[3].set(W2r.T @ wb))
  cvec8 = (jnp.zeros((1, 8), _f32)
           .at[0, 0].set(b2l @ wa + bp[0])
           .at[0, 1].set(b2l @ wb))

  PP = _pad_to(PE, 128)
  TE = _pad_to(PP + NE, NW * 8 * 128)
  fill = (jnp.arange(TE, dtype=jnp.int32) * 89) % N
  qsrc = fill.at[:PE].set(pos_edge_index[0]).at[PP:PP + NE].set(neg_edge_index[0])
  qdst = fill.at[:PE].set(pos_edge_index[1]).at[PP:PP + NE].set(neg_edge_index[1])
  qsrc = qsrc.reshape(TE // 128, 128)
  qdst = (qdst + NP).reshape(TE // 128, 128)

  zr2 = jnp.zeros((NP // NS, D), _f32)
  zr1 = jnp.zeros((NP // NS,), _f32)
  on1 = jnp.ones((128,), _f32)

  edge_pass = _make_edge_pass(NP, D, CPT, with_deg=True)
  edge_narrow = _make_edge_narrow(NP, CPT)
  dense1 = _make_dense1(NP, D, H, BM)
  dense2 = _make_dense2(NP, BM)
  decode = _make_decode(2 * NP, TE)
  NB = NP // BM

  p, dg = edge_pass(xp, src, dst, zr2, zr1, on1)
  d3 = dg.reshape(NC * NB, 1, BM)
  z1, proj8 = dense1(p, p, d3, d3, xp, W1lT, W1rT, b1, U8)

  pa, pb = edge_narrow(proj8, src, dst, zr1)
  pa3 = pa.reshape(NC * NB, 1, BM)
  pb3 = pb.reshape(NC * NB, 1, BM)
  ab8 = dense2(pa3, pa3, pb3, pb3, d3, d3, proj8, cvec8)

  dec = decode(ab8[:2].reshape(2 * NP), qsrc, qdst)
  pos = dec[:PE]
  neg = dec[PP:PP + NE]
  return (pos, neg)


# speedup vs baseline: 18.5912x; 1.0226x over previous
import functools

import jax
import jax.numpy as jnp
from jax import lax
from jax.experimental import pallas as pl
from jax.experimental.pallas import tpu as pltpu
from jax.experimental.pallas import tpu_sc as plsc

NC = 2
NS = 16
NW = NC * NS

_f32 = jnp.float32


def _make_edge_pass(NP, D, CPT, G=16, with_deg=True):
  RPT = NP // NS
  mesh = plsc.VectorSubcoreMesh(core_axis_name="c", subcore_axis_name="s",
                                num_cores=NC, num_subcores=NS)

  out_type = [jax.ShapeDtypeStruct((NC * NP, D), _f32)]
  if with_deg:
    out_type.append(jax.ShapeDtypeStruct((NC * NP,), _f32))

  @functools.partial(
      pl.kernel,
      out_type=tuple(out_type),
      mesh=mesh,
      scratch_types=[
          pltpu.VMEM((G, 128), jnp.int32),
          pltpu.VMEM((G, 128), jnp.int32),
          pltpu.VMEM((128, D), _f32),
          pltpu.VMEM((128, D), _f32),
          pltpu.VMEM((128,), _f32),
          pltpu.VMEM_SHARED((NP, D), _f32),
          pltpu.VMEM_SHARED((NP,), _f32),
          pltpu.SemaphoreType.DMA,
          pltpu.SemaphoreType.DMA,
          pltpu.SemaphoreType.DMA,
          pltpu.SemaphoreType.DMA,
      ],
  )
  def edge_pass(x_hbm, src_hbm, dst_hbm, z2d_hbm, z1d_hbm, on_hbm, outp_hbm,
                *refs):
    if with_deg:
      outd_hbm = refs[0]
      refs = refs[1:]
    sidx, didx, rowsa, rowsb, ones, acc, deg, gsa, gsb, ssa, ssb = refs
    c = lax.axis_index("c")
    s = lax.axis_index("s")
    wid = c * NS + s

    base = s * RPT
    pltpu.sync_copy(z2d_hbm, acc.at[pl.ds(base, RPT)])
    if with_deg:
      pltpu.sync_copy(z1d_hbm, deg.at[pl.ds(base, RPT)])
      pltpu.sync_copy(on_hbm, ones)
    plsc.subcore_barrier()

    def gloop(g, _):
      gb_ = wid * CPT + g * G
      pltpu.sync_copy(src_hbm.at[pl.ds(gb_, G)], sidx)
      pltpu.sync_copy(dst_hbm.at[pl.ds(gb_, G)], didx)
      pltpu.make_async_copy(x_hbm.at[sidx.at[0]], rowsa, gsa).start()
      pltpu.make_async_copy(x_hbm.at[sidx.at[1]], rowsb, gsb).start()

      def eloop(jj, _):
        j0 = 2 * jj
        j1 = j0 + 1
        pltpu.make_async_copy(x_hbm.at[sidx.at[j0]], rowsa, gsa).wait()
        sca = pltpu.make_async_copy(rowsa, acc.at[didx.at[j0]], ssa)
        sca.start(add=True)
        if with_deg:
          pltpu.sync_copy(ones, deg.at[didx.at[j0]], add=True)
        pltpu.make_async_copy(x_hbm.at[sidx.at[j1]], rowsb, gsb).wait()
        scb = pltpu.make_async_copy(rowsb, acc.at[didx.at[j1]], ssb)
        scb.start(add=True)
        if with_deg:
          pltpu.sync_copy(ones, deg.at[didx.at[j1]], add=True)
        sca.wait()

        @pl.when(j0 + 2 < G)
        def _():
          pltpu.make_async_copy(x_hbm.at[sidx.at[j0 + 2]], rowsa, gsa).start()
        scb.wait()

        @pl.when(j1 + 2 < G)
        def _():
          pltpu.make_async_copy(x_hbm.at[sidx.at[j1 + 2]], rowsb, gsb).start()
        return 0
      lax.fori_loop(0, G // 2, eloop, 0)
      return 0
    lax.fori_loop(0, CPT // G, gloop, 0)

    plsc.subcore_barrier()

    pltpu.sync_copy(acc.at[pl.ds(base, RPT)],
                    outp_hbm.at[pl.ds(c * NP + base, RPT)])
    if with_deg:
      pltpu.sync_copy(deg.at[pl.ds(base, RPT)],
                      outd_hbm.at[pl.ds(c * NP + base, RPT)])

  return edge_pass


def _make_dense1(NP, D, H, BM):
  NB = NP // BM

  def body(p0_ref, p1_ref, d0_ref, d1_ref, x_ref, wl_ref, wr_ref, b_ref,
           u_ref, o_ref, pr_ref):
    d = d0_ref[0, 0, :] + d1_ref[0, 0, :]
    inv = 1.0 / jnp.maximum(d, 1.0)
    mean = (p0_ref[...] + p1_ref[...]) * inv[:, None]
    z = (jnp.dot(mean, wl_ref[...], preferred_element_type=_f32)
         + jnp.dot(x_ref[...], wr_ref[...], preferred_element_type=_f32)
         + b_ref[...])
    z = jnp.maximum(z, 0.0)
    o_ref[...] = z
    pr_ref[...] = jax.lax.dot_general(
        u_ref[...], z, (((1,), (1,)), ((), ())),
        preferred_element_type=_f32)

  return pl.pallas_call(
      body,
      grid=(NB,),
      in_specs=[
          pl.BlockSpec((BM, D), lambda i: (i, 0)),
          pl.BlockSpec((BM, D), lambda i: (i + NB, 0)),
          pl.BlockSpec((1, 1, BM), lambda i: (i, 0, 0)),
          pl.BlockSpec((1, 1, BM), lambda i: (i + NB, 0, 0)),
          pl.BlockSpec((BM, D), lambda i: (i, 0)),
          pl.BlockSpec((D, H), lambda i: (0, 0)),
          pl.BlockSpec((D, H), lambda i: (0, 0)),
          pl.BlockSpec((1, H), lambda i: (0, 0)),
          pl.BlockSpec((8, H), lambda i: (0, 0)),
      ],
      out_specs=[
          pl.BlockSpec((BM, H), lambda i: (i, 0)),
          pl.BlockSpec((8, BM), lambda i: (0, i)),
      ],
      out_shape=[
          jax.ShapeDtypeStruct((NP, H), _f32),
          jax.ShapeDtypeStruct((8, NP), _f32),
      ],
  )


def _make_edge_narrow(NP, CPT, GQ=8):
  RPT = NP // NS
  mesh = plsc.VectorSubcoreMesh(core_axis_name="c", subcore_axis_name="s",
                                num_cores=NC, num_subcores=NS)

  @functools.partial(
      pl.kernel,
      out_type=(
          jax.ShapeDtypeStruct((NC * NP,), _f32),
          jax.ShapeDtypeStruct((NC * NP,), _f32),
      ),
      mesh=mesh,
      scratch_types=[
          pltpu.VMEM((CPT, 128), jnp.int32),
          pltpu.VMEM((CPT, 128), jnp.int32),
          pltpu.VMEM((GQ * 128,), _f32),
          pltpu.VMEM((GQ * 128,), _f32),
          pltpu.VMEM_SHARED((NP,), _f32),
          pltpu.VMEM_SHARED((NP,), _f32),
          pltpu.VMEM_SHARED((NP,), _f32),
          pltpu.VMEM_SHARED((NP,), _f32),
          pltpu.SemaphoreType.DMA,
          pltpu.SemaphoreType.DMA,
          pltpu.SemaphoreType.DMA,
          pltpu.SemaphoreType.DMA,
      ],
  )
  def narrow(t_hbm, src_hbm, dst_hbm, z1d_hbm, outa_hbm, outb_hbm,
             si, di, bufa, bufb, ash, bsh, acca, accb, sma, smb, swa, swb):
    c = lax.axis_index("c")
    s = lax.axis_index("s")
    wid = c * NS + s
    base = s * RPT

    @pl.when(s == 0)
    def _():
      pltpu.sync_copy(t_hbm.at[0], ash)
      pltpu.sync_copy(t_hbm.at[1], bsh)
    pltpu.sync_copy(z1d_hbm, acca.at[pl.ds(base, RPT)])
    pltpu.sync_copy(z1d_hbm, accb.at[pl.ds(base, RPT)])
    pltpu.sync_copy(src_hbm.at[pl.ds(wid * CPT, CPT)], si)
    pltpu.sync_copy(dst_hbm.at[pl.ds(wid * CPT, CPT)], di)
    plsc.subcore_barrier()

    def loop(g, _):
      j0 = g * GQ

      @pl.when(g > 0)
      def _():
        for k in range(GQ):
          jp = j0 - GQ + k
          pltpu.make_async_copy(bufa.at[pl.ds(k * 128, 128)],
                                acca.at[di.at[jp]], swa).wait()
          pltpu.make_async_copy(bufb.at[pl.ds(k * 128, 128)],
                                accb.at[di.at[jp]], swb).wait()
      for k in range(GQ):
        pltpu.make_async_copy(ash.at[si.at[j0 + k]],
                              bufa.at[pl.ds(k * 128, 128)], sma).start()
        pltpu.make_async_copy(bsh.at[si.at[j0 + k]],
                              bufb.at[pl.ds(k * 128, 128)], smb).start()
      for k in range(GQ):
        pltpu.make_async_copy(ash.at[si.at[j0 + k]],
                              bufa.at[pl.ds(k * 128, 128)], sma).wait()
        pltpu.make_async_copy(bsh.at[si.at[j0 + k]],
                              bufb.at[pl.ds(k * 128, 128)], smb).wait()
      for k in range(GQ):
        pltpu.make_async_copy(bufa.at[pl.ds(k * 128, 128)],
                              acca.at[di.at[j0 + k]], swa).start(add=True)
        pltpu.make_async_copy(bufb.at[pl.ds(k * 128, 128)],
                              accb.at[di.at[j0 + k]], swb).start(add=True)
      return 0
    lax.fori_loop(0, CPT // GQ, loop, 0)

    for k in range(GQ):
      jp = CPT - GQ + k
      pltpu.make_async_copy(bufa.at[pl.ds(k * 128, 128)],
                            acca.at[di.at[jp]], swa).wait()
      pltpu.make_async_copy(bufb.at[pl.ds(k * 128, 128)],
                            accb.at[di.at[jp]], swb).wait()
    plsc.subcore_barrier()

    pltpu.sync_copy(acca.at[pl.ds(base, RPT)],
                    outa_hbm.at[pl.ds(c * NP + base, RPT)])
    pltpu.sync_copy(accb.at[pl.ds(base, RPT)],
                    outb_hbm.at[pl.ds(c * NP + base, RPT)])

  return narrow


def _make_decode(NP, TE, GQ=8):
  EPT = TE // NW
  CQ = EPT // 128
  RPT = NP // NS
  mesh = plsc.VectorSubcoreMesh(core_axis_name="c", subcore_axis_name="s",
                                num_cores=NC, num_subcores=NS)

  @functools.partial(
      pl.kernel,
      out_type=jax.ShapeDtypeStruct((TE,), _f32),
      mesh=mesh,
      scratch_types=[
          pltpu.VMEM((CQ, 128), jnp.int32),
          pltpu.VMEM((CQ, 128), jnp.int32),
          pltpu.VMEM((GQ * 128,), _f32),
          pltpu.VMEM((GQ * 128,), _f32),
          pltpu.VMEM((EPT,), _f32),
          pltpu.VMEM((RPT,), _f32),
          pltpu.VMEM((RPT,), _f32),
          pltpu.VMEM((RPT,), _f32),
          pltpu.VMEM((RPT,), _f32),
          pltpu.VMEM((RPT,), _f32),
          pltpu.VMEM((RPT,), _f32),
          pltpu.VMEM((RPT,), _f32),
          pltpu.VMEM((RPT,), _f32),
          pltpu.VMEM((RPT,), _f32),
          pltpu.VMEM((RPT,), _f32),
          pltpu.VMEM_SHARED((NP,), _f32),
          pltpu.VMEM_SHARED((NP,), _f32),
          pltpu.SemaphoreType.DMA,
          pltpu.SemaphoreType.DMA,
      ],
  )
  def decode(pa_hbm, pb_hbm, dg_hbm, ga_hbm, gb_hbm, s_hbm, d_hbm, out_hbm,
             si, di, bufa, bufb, ov, a0, a1, b0, b1, dd0, dd1, gga, ggb,
             ta, tb, ash, bsh, sma, smb):
    c = lax.axis_index("c")
    s = lax.axis_index("s")
    wid = c * NS + s
    base = s * RPT
    pltpu.sync_copy(pa_hbm.at[pl.ds(base, RPT)], a0)
    pltpu.sync_copy(pa_hbm.at[pl.ds(NP + base, RPT)], a1)
    pltpu.sync_copy(pb_hbm.at[pl.ds(base, RPT)], b0)
    pltpu.sync_copy(pb_hbm.at[pl.ds(NP + base, RPT)], b1)
    pltpu.sync_copy(dg_hbm.at[pl.ds(base, RPT)], dd0)
    pltpu.sync_copy(dg_hbm.at[pl.ds(NP + base, RPT)], dd1)
    pltpu.sync_copy(ga_hbm.at[pl.ds(base, RPT)], gga)
    pltpu.sync_copy(gb_hbm.at[pl.ds(base, RPT)], ggb)
    pltpu.sync_copy(s_hbm.at[pl.ds(wid * CQ, CQ)], si)
    pltpu.sync_copy(d_hbm.at[pl.ds(wid * CQ, CQ)], di)

    def build(i, _):
      o = i * 16
      sl = pl.ds(o, 16)
      inv = 1.0 / jnp.maximum(dd0[sl] + dd1[sl], 1.0)
      ta[sl] = (a0[sl] + a1[sl]) * inv + gga[sl]
      tb[sl] = (b0[sl] + b1[sl]) * inv + ggb[sl]
      return 0
    lax.fori_loop(0, RPT // 16, build, 0)
    pltpu.sync_copy(ta, ash.at[pl.ds(base, RPT)])
    pltpu.sync_copy(tb, bsh.at[pl.ds(base, RPT)])
    plsc.subcore_barrier()

    def loop(g, _):
      j0 = g * GQ
      for k in range(GQ):
        pltpu.make_async_copy(ash.at[si.at[j0 + k]],
                              bufa.at[pl.ds(k * 128, 128)], sma).start()
        pltpu.make_async_copy(bsh.at[di.at[j0 + k]],
                              bufb.at[pl.ds(k * 128, 128)], smb).start()
      for k in range(GQ):
        pltpu.make_async_copy(ash.at[si.at[j0 + k]],
                              bufa.at[pl.ds(k * 128, 128)], sma).wait()
        pltpu.make_async_copy(bsh.at[di.at[j0 + k]],
                              bufb.at[pl.ds(k * 128, 128)], smb).wait()
      for k in range(GQ * 128 // 16):
        o = k * 16
        ov[pl.ds(j0 * 128 + o, 16)] = (bufa[pl.ds(o, 16)]
                                       + bufb[pl.ds(o, 16)])
      return 0
    lax.fori_loop(0, CQ // GQ, loop, 0)

    pltpu.sync_copy(ov, out_hbm.at[pl.ds(wid * EPT, EPT)])

  return decode


def _pad_to(v, m):
  return ((v + m - 1) // m) * m


def kernel(x, edge_index, edge_weight, pos_edge_index, neg_edge_index,
           W1l, b1l, W1r, W2l, b2l, W2r, Wp, bp):
  N, D = x.shape
  H = W1l.shape[0]
  E = edge_index.shape[1]
  PE = pos_edge_index.shape[1]
  NE = neg_edge_index.shape[1]

  BM = 512
  NP = _pad_to(N, max(BM, NS * 128))
  EP = _pad_to(E, NW * 256)
  CPT = EP // (128 * NW)

  xp = jnp.zeros((NP, D), _f32).at[:N].set(x)

  npad = EP - E
  pad_src = (jnp.arange(npad, dtype=jnp.int32) * 97) % N
  pad_dst = N + (jnp.arange(npad, dtype=jnp.int32) % (NP - N))
  src = jnp.concatenate([edge_index[0], pad_src]).reshape(EP // 128, 128)
  dst = jnp.concatenate([edge_index[1], pad_dst]).reshape(EP // 128, 128)

  W1lT = W1l.T
  W1rT = W1r.T
  b1 = b1l.reshape(1, H)

  wa = Wp[0, :H]
  wb = Wp[0, H:]
  U8 = (jnp.zeros((8, H), _f32)
        .at[0].set(W2l.T @ wa).at[1].set(W2l.T @ wb)
        .at[2].set(W2r.T @ wa).at[3].set(W2r.T @ wb))
  cvec8 = (jnp.zeros((1, 8), _f32)
           .at[0, 0].set(b2l @ wa + bp[0])
           .at[0, 1].set(b2l @ wb))

  PP = _pad_to(PE, 128)
  TE = _pad_to(PP + NE, NW * 8 * 128)
  fill = (jnp.arange(TE, dtype=jnp.int32) * 89) % N
  qsrc = fill.at[:PE].set(pos_edge_index[0]).at[PP:PP + NE].set(neg_edge_index[0])
  qdst = fill.at[:PE].set(pos_edge_index[1]).at[PP:PP + NE].set(neg_edge_index[1])
  qsrc = qsrc.reshape(TE // 128, 128)
  qdst = qdst.reshape(TE // 128, 128)

  zr2 = jnp.zeros((NP // NS, D), _f32)
  zr1 = jnp.zeros((NP // NS,), _f32)
  on1 = jnp.ones((128,), _f32)

  edge_pass = _make_edge_pass(NP, D, CPT, with_deg=True)
  edge_narrow = _make_edge_narrow(NP, CPT)
  dense1 = _make_dense1(NP, D, H, BM)
  decode = _make_decode(NP, TE)
  NB = NP // BM

  p, dg = edge_pass(xp, src, dst, zr2, zr1, on1)
  d3 = dg.reshape(NC * NB, 1, BM)
  z1, proj8 = dense1(p, p, d3, d3, xp, W1lT, W1rT, b1, U8)

  pa, pb = edge_narrow(proj8, src, dst, zr1)

  ga = proj8[2] + cvec8[0, 0]
  gb = proj8[3] + cvec8[0, 1]
  dec = decode(pa, pb, dg, ga, gb, qsrc, qdst)
  pos = dec[:PE]
  neg = dec[PP:PP + NE]
  return (pos, neg)


# speedup vs baseline: 18.6935x; 1.0055x over previous
import functools

import jax
import jax.numpy as jnp
from jax import lax
from jax.experimental import pallas as pl
from jax.experimental.pallas import tpu as pltpu
from jax.experimental.pallas import tpu_sc as plsc

NC = 2
NS = 16
NW = NC * NS

_f32 = jnp.float32


def _make_edge_pass(NP, D, CPT, G=16, with_deg=True):
  RPT = NP // NS
  mesh = plsc.VectorSubcoreMesh(core_axis_name="c", subcore_axis_name="s",
                                num_cores=NC, num_subcores=NS)

  out_type = [jax.ShapeDtypeStruct((NC * NP, D), _f32)]
  if with_deg:
    out_type.append(jax.ShapeDtypeStruct((NC * NP,), _f32))

  @functools.partial(
      pl.kernel,
      out_type=tuple(out_type),
      mesh=mesh,
      scratch_types=[
          pltpu.VMEM((G, 128), jnp.int32),
          pltpu.VMEM((G, 128), jnp.int32),
          pltpu.VMEM((128, D), _f32),
          pltpu.VMEM((128, D), _f32),
          pltpu.VMEM((128,), _f32),
          pltpu.VMEM_SHARED((NP, D), _f32),
          pltpu.VMEM_SHARED((NP,), _f32),
          pltpu.SemaphoreType.DMA,
          pltpu.SemaphoreType.DMA,
          pltpu.SemaphoreType.DMA,
          pltpu.SemaphoreType.DMA,
      ],
  )
  def edge_pass(x_hbm, src_hbm, dst_hbm, z2d_hbm, z1d_hbm, on_hbm, outp_hbm,
                *refs):
    if with_deg:
      outd_hbm = refs[0]
      refs = refs[1:]
    sidx, didx, rowsa, rowsb, ones, acc, deg, gsa, gsb, ssa, ssb = refs
    c = lax.axis_index("c")
    s = lax.axis_index("s")
    wid = c * NS + s

    base = s * RPT
    pltpu.sync_copy(z2d_hbm, acc.at[pl.ds(base, RPT)])
    if with_deg:
      pltpu.sync_copy(z1d_hbm, deg.at[pl.ds(base, RPT)])
      pltpu.sync_copy(on_hbm, ones)
    plsc.subcore_barrier()

    def gloop(g, _):
      gb_ = wid * CPT + g * G
      pltpu.sync_copy(src_hbm.at[pl.ds(gb_, G)], sidx)
      pltpu.sync_copy(dst_hbm.at[pl.ds(gb_, G)], didx)
      pltpu.make_async_copy(x_hbm.at[sidx.at[0]], rowsa, gsa).start()
      pltpu.make_async_copy(x_hbm.at[sidx.at[1]], rowsb, gsb).start()

      def eloop(jj, _):
        j0 = 2 * jj
        j1 = j0 + 1
        pltpu.make_async_copy(x_hbm.at[sidx.at[j0]], rowsa, gsa).wait()
        sca = pltpu.make_async_copy(rowsa, acc.at[didx.at[j0]], ssa)
        sca.start(add=True)
        if with_deg:
          pltpu.sync_copy(ones, deg.at[didx.at[j0]], add=True)
        pltpu.make_async_copy(x_hbm.at[sidx.at[j1]], rowsb, gsb).wait()
        scb = pltpu.make_async_copy(rowsb, acc.at[didx.at[j1]], ssb)
        scb.start(add=True)
        if with_deg:
          pltpu.sync_copy(ones, deg.at[didx.at[j1]], add=True)
        sca.wait()

        @pl.when(j0 + 2 < G)
        def _():
          pltpu.make_async_copy(x_hbm.at[sidx.at[j0 + 2]], rowsa, gsa).start()
        scb.wait()

        @pl.when(j1 + 2 < G)
        def _():
          pltpu.make_async_copy(x_hbm.at[sidx.at[j1 + 2]], rowsb, gsb).start()
        return 0
      lax.fori_loop(0, G // 2, eloop, 0)
      return 0
    lax.fori_loop(0, CPT // G, gloop, 0)

    plsc.subcore_barrier()

    pltpu.sync_copy(acc.at[pl.ds(base, RPT)],
                    outp_hbm.at[pl.ds(c * NP + base, RPT)])
    if with_deg:
      pltpu.sync_copy(deg.at[pl.ds(base, RPT)],
                      outd_hbm.at[pl.ds(c * NP + base, RPT)])

  return edge_pass


def _make_dense1(NP, D, H, BM):
  NB = NP // BM

  def body(p0_ref, p1_ref, d0_ref, d1_ref, x_ref, wl_ref, wr_ref, b_ref,
           u_ref, pr_ref):
    d = d0_ref[0, 0, :] + d1_ref[0, 0, :]
    inv = 1.0 / jnp.maximum(d, 1.0)
    mean = (p0_ref[...] + p1_ref[...]) * inv[:, None]
    z = (jnp.dot(mean, wl_ref[...], preferred_element_type=_f32)
         + jnp.dot(x_ref[...], wr_ref[...], preferred_element_type=_f32)
         + b_ref[...])
    z = jnp.maximum(z, 0.0)
    pr_ref[...] = jax.lax.dot_general(
        u_ref[...], z, (((1,), (1,)), ((), ())),
        preferred_element_type=_f32)

  return pl.pallas_call(
      body,
      grid=(NB,),
      in_specs=[
          pl.BlockSpec((BM, D), lambda i: (i, 0)),
          pl.BlockSpec((BM, D), lambda i: (i + NB, 0)),
          pl.BlockSpec((1, 1, BM), lambda i: (i, 0, 0)),
          pl.BlockSpec((1, 1, BM), lambda i: (i + NB, 0, 0)),
          pl.BlockSpec((BM, D), lambda i: (i, 0)),
          pl.BlockSpec((D, H), lambda i: (0, 0)),
          pl.BlockSpec((D, H), lambda i: (0, 0)),
          pl.BlockSpec((1, H), lambda i: (0, 0)),
          pl.BlockSpec((8, H), lambda i: (0, 0)),
      ],
      out_specs=pl.BlockSpec((8, BM), lambda i: (0, i)),
      out_shape=jax.ShapeDtypeStruct((8, NP), _f32),
  )


def _make_edge_narrow(NP, CPT, GQ=8):
  RPT = NP // NS
  mesh = plsc.VectorSubcoreMesh(core_axis_name="c", subcore_axis_name="s",
                                num_cores=NC, num_subcores=NS)

  @functools.partial(
      pl.kernel,
      out_type=(
          jax.ShapeDtypeStruct((NC * NP,), _f32),
          jax.ShapeDtypeStruct((NC * NP,), _f32),
      ),
      mesh=mesh,
      scratch_types=[
          pltpu.VMEM((CPT, 128), jnp.int32),
          pltpu.VMEM((CPT, 128), jnp.int32),
          pltpu.VMEM((GQ * 128,), _f32),
          pltpu.VMEM((GQ * 128,), _f32),
          pltpu.VMEM_SHARED((NP,), _f32),
          pltpu.VMEM_SHARED((NP,), _f32),
          pltpu.VMEM_SHARED((NP,), _f32),
          pltpu.VMEM_SHARED((NP,), _f32),
          pltpu.SemaphoreType.DMA,
          pltpu.SemaphoreType.DMA,
          pltpu.SemaphoreType.DMA,
          pltpu.SemaphoreType.DMA,
      ],
  )
  def narrow(t_hbm, src_hbm, dst_hbm, z1d_hbm, outa_hbm, outb_hbm,
             si, di, bufa, bufb, ash, bsh, acca, accb, sma, smb, swa, swb):
    c = lax.axis_index("c")
    s = lax.axis_index("s")
    wid = c * NS + s
    base = s * RPT

    @pl.when(s == 0)
    def _():
      pltpu.sync_copy(t_hbm.at[0], ash)
      pltpu.sync_copy(t_hbm.at[1], bsh)
    pltpu.sync_copy(z1d_hbm, acca.at[pl.ds(base, RPT)])
    pltpu.sync_copy(z1d_hbm, accb.at[pl.ds(base, RPT)])
    pltpu.sync_copy(src_hbm.at[pl.ds(wid * CPT, CPT)], si)
    pltpu.sync_copy(dst_hbm.at[pl.ds(wid * CPT, CPT)], di)
    plsc.subcore_barrier()

    def loop(g, _):
      j0 = g * GQ

      @pl.when(g > 0)
      def _():
        for k in range(GQ):
          jp = j0 - GQ + k
          pltpu.make_async_copy(bufa.at[pl.ds(k * 128, 128)],
                                acca.at[di.at[jp]], swa).wait()
          pltpu.make_async_copy(bufb.at[pl.ds(k * 128, 128)],
                                accb.at[di.at[jp]], swb).wait()
      for k in range(GQ):
        pltpu.make_async_copy(ash.at[si.at[j0 + k]],
                              bufa.at[pl.ds(k * 128, 128)], sma).start()
        pltpu.make_async_copy(bsh.at[si.at[j0 + k]],
                              bufb.at[pl.ds(k * 128, 128)], smb).start()
      for k in range(GQ):
        pltpu.make_async_copy(ash.at[si.at[j0 + k]],
                              bufa.at[pl.ds(k * 128, 128)], sma).wait()
        pltpu.make_async_copy(bsh.at[si.at[j0 + k]],
                              bufb.at[pl.ds(k * 128, 128)], smb).wait()
      for k in range(GQ):
        pltpu.make_async_copy(bufa.at[pl.ds(k * 128, 128)],
                              acca.at[di.at[j0 + k]], swa).start(add=True)
        pltpu.make_async_copy(bufb.at[pl.ds(k * 128, 128)],
                              accb.at[di.at[j0 + k]], swb).start(add=True)
      return 0
    lax.fori_loop(0, CPT // GQ, loop, 0)

    for k in range(GQ):
      jp = CPT - GQ + k
      pltpu.make_async_copy(bufa.at[pl.ds(k * 128, 128)],
                            acca.at[di.at[jp]], swa).wait()
      pltpu.make_async_copy(bufb.at[pl.ds(k * 128, 128)],
                            accb.at[di.at[jp]], swb).wait()
    plsc.subcore_barrier()

    pltpu.sync_copy(acca.at[pl.ds(base, RPT)],
                    outa_hbm.at[pl.ds(c * NP + base, RPT)])
    pltpu.sync_copy(accb.at[pl.ds(base, RPT)],
                    outb_hbm.at[pl.ds(c * NP + base, RPT)])

  return narrow


def _make_decode(NP, TE, GQ=8):
  EPT = TE // NW
  CQ = EPT // 128
  RPT = NP // NS
  mesh = plsc.VectorSubcoreMesh(core_axis_name="c", subcore_axis_name="s",
                                num_cores=NC, num_subcores=NS)

  @functools.partial(
      pl.kernel,
      out_type=jax.ShapeDtypeStruct((TE,), _f32),
      mesh=mesh,
      scratch_types=[
          pltpu.VMEM((CQ, 128), jnp.int32),
          pltpu.VMEM((CQ, 128), jnp.int32),
          pltpu.VMEM((GQ * 128,), _f32),
          pltpu.VMEM((GQ * 128,), _f32),
          pltpu.VMEM((EPT,), _f32),
          pltpu.VMEM((RPT,), _f32),
          pltpu.VMEM((RPT,), _f32),
          pltpu.VMEM((RPT,), _f32),
          pltpu.VMEM((RPT,), _f32),
          pltpu.VMEM((RPT,), _f32),
          pltpu.VMEM((RPT,), _f32),
          pltpu.VMEM((RPT,), _f32),
          pltpu.VMEM((RPT,), _f32),
          pltpu.VMEM((RPT,), _f32),
          pltpu.VMEM((RPT,), _f32),
          pltpu.VMEM_SHARED((NP,), _f32),
          pltpu.VMEM_SHARED((NP,), _f32),
          pltpu.SemaphoreType.DMA,
          pltpu.SemaphoreType.DMA,
      ],
  )
  def decode(pa_hbm, pb_hbm, dg_hbm, ga_hbm, gb_hbm, s_hbm, d_hbm, out_hbm,
             si, di, bufa, bufb, ov, a0, a1, b0, b1, dd0, dd1, gga, ggb,
             ta, tb, ash, bsh, sma, smb):
    c = lax.axis_index("c")
    s = lax.axis_index("s")
    wid = c * NS + s
    base = s * RPT
    pltpu.sync_copy(pa_hbm.at[pl.ds(base, RPT)], a0)
    pltpu.sync_copy(pa_hbm.at[pl.ds(NP + base, RPT)], a1)
    pltpu.sync_copy(pb_hbm.at[pl.ds(base, RPT)], b0)
    pltpu.sync_copy(pb_hbm.at[pl.ds(NP + base, RPT)], b1)
    pltpu.sync_copy(dg_hbm.at[pl.ds(base, RPT)], dd0)
    pltpu.sync_copy(dg_hbm.at[pl.ds(NP + base, RPT)], dd1)
    pltpu.sync_copy(ga_hbm.at[pl.ds(base, RPT)], gga)
    pltpu.sync_copy(gb_hbm.at[pl.ds(base, RPT)], ggb)
    pltpu.sync_copy(s_hbm.at[pl.ds(wid * CQ, CQ)], si)
    pltpu.sync_copy(d_hbm.at[pl.ds(wid * CQ, CQ)], di)

    def build(i, _):
      o = i * 16
      sl = pl.ds(o, 16)
      inv = 1.0 / jnp.maximum(dd0[sl] + dd1[sl], 1.0)
      ta[sl] = (a0[sl] + a1[sl]) * inv + gga[sl]
      tb[sl] = (b0[sl] + b1[sl]) * inv + ggb[sl]
      return 0
    lax.fori_loop(0, RPT // 16, build, 0)
    pltpu.sync_copy(ta, ash.at[pl.ds(base, RPT)])
    pltpu.sync_copy(tb, bsh.at[pl.ds(base, RPT)])
    plsc.subcore_barrier()

    def loop(g, _):
      j0 = g * GQ
      for k in range(GQ):
        pltpu.make_async_copy(ash.at[si.at[j0 + k]],
                              bufa.at[pl.ds(k * 128, 128)], sma).start()
        pltpu.make_async_copy(bsh.at[di.at[j0 + k]],
                              bufb.at[pl.ds(k * 128, 128)], smb).start()
      for k in range(GQ):
        pltpu.make_async_copy(ash.at[si.at[j0 + k]],
                              bufa.at[pl.ds(k * 128, 128)], sma).wait()
        pltpu.make_async_copy(bsh.at[di.at[j0 + k]],
                              bufb.at[pl.ds(k * 128, 128)], smb).wait()
      for k in range(GQ * 128 // 16):
        o = k * 16
        ov[pl.ds(j0 * 128 + o, 16)] = (bufa[pl.ds(o, 16)]
                                       + bufb[pl.ds(o, 16)])
      return 0
    lax.fori_loop(0, CQ // GQ, loop, 0)

    pltpu.sync_copy(ov, out_hbm.at[pl.ds(wid * EPT, EPT)])

  return decode


def _pad_to(v, m):
  return ((v + m - 1) // m) * m


def kernel(x, edge_index, edge_weight, pos_edge_index, neg_edge_index,
           W1l, b1l, W1r, W2l, b2l, W2r, Wp, bp):
  N, D = x.shape
  H = W1l.shape[0]
  E = edge_index.shape[1]
  PE = pos_edge_index.shape[1]
  NE = neg_edge_index.shape[1]

  BM = 512
  NP = _pad_to(N, max(BM, NS * 128))
  EP = _pad_to(E, NW * 256)
  CPT = EP // (128 * NW)

  xp = jnp.zeros((NP, D), _f32).at[:N].set(x)

  npad = EP - E
  pad_src = (jnp.arange(npad, dtype=jnp.int32) * 97) % N
  pad_dst = N + (jnp.arange(npad, dtype=jnp.int32) % (NP - N))
  src = jnp.concatenate([edge_index[0], pad_src]).reshape(EP // 128, 128)
  dst = jnp.concatenate([edge_index[1], pad_dst]).reshape(EP // 128, 128)

  W1lT = W1l.T
  W1rT = W1r.T
  b1 = b1l.reshape(1, H)

  wa = Wp[0, :H]
  wb = Wp[0, H:]
  U8 = (jnp.zeros((8, H), _f32)
        .at[0].set(W2l.T @ wa).at[1].set(W2l.T @ wb)
        .at[2].set(W2r.T @ wa).at[3].set(W2r.T @ wb))
  cvec8 = (jnp.zeros((1, 8), _f32)
           .at[0, 0].set(b2l @ wa + bp[0])
           .at[0, 1].set(b2l @ wb))

  PP = _pad_to(PE, 128)
  TE = _pad_to(PP + NE, NW * 8 * 128)
  fill = (jnp.arange(TE, dtype=jnp.int32) * 89) % N
  qsrc = fill.at[:PE].set(pos_edge_index[0]).at[PP:PP + NE].set(neg_edge_index[0])
  qdst = fill.at[:PE].set(pos_edge_index[1]).at[PP:PP + NE].set(neg_edge_index[1])
  qsrc = qsrc.reshape(TE // 128, 128)
  qdst = qdst.reshape(TE // 128, 128)

  zr2 = jnp.zeros((NP // NS, D), _f32)
  zr1 = jnp.zeros((NP // NS,), _f32)
  on1 = jnp.ones((128,), _f32)

  edge_pass = _make_edge_pass(NP, D, CPT, with_deg=True)
  edge_narrow = _make_edge_narrow(NP, CPT)
  dense1 = _make_dense1(NP, D, H, BM)
  decode = _make_decode(NP, TE)
  NB = NP // BM

  p, dg = edge_pass(xp, src, dst, zr2, zr1, on1)
  d3 = dg.reshape(NC * NB, 1, BM)
  proj8 = dense1(p, p, d3, d3, xp, W1lT, W1rT, b1, U8)

  pa, pb = edge_narrow(proj8, src, dst, zr1)

  ga = proj8[2] + cvec8[0, 0]
  gb = proj8[3] + cvec8[0, 1]
  dec = decode(pa, pb, dg, ga, gb, qsrc, qdst)
  pos = dec[:PE]
  neg = dec[PP:PP + NE]
  return (pos, neg)


# speedup vs baseline: 18.9321x; 1.0128x over previous
import functools

import jax
import jax.numpy as jnp
from jax import lax
from jax.experimental import pallas as pl
from jax.experimental.pallas import tpu as pltpu
from jax.experimental.pallas import tpu_sc as plsc

NC = 2
NS = 16
NW = NC * NS

_f32 = jnp.float32


def _make_edge_pass(NP, D, CPT, G=16, with_deg=True):
  RPT = NP // NS
  mesh = plsc.VectorSubcoreMesh(core_axis_name="c", subcore_axis_name="s",
                                num_cores=NC, num_subcores=NS)

  out_type = [jax.ShapeDtypeStruct((NC * NP, D), _f32)]
  if with_deg:
    out_type.append(jax.ShapeDtypeStruct((NC * NP,), _f32))

  @functools.partial(
      pl.kernel,
      out_type=tuple(out_type),
      mesh=mesh,
      scratch_types=[
          pltpu.VMEM((G, 128), jnp.int32),
          pltpu.VMEM((G, 128), jnp.int32),
          pltpu.VMEM((128, D), _f32),
          pltpu.VMEM((128, D), _f32),
          pltpu.VMEM((128,), _f32),
          pltpu.VMEM_SHARED((NP, D), _f32),
          pltpu.VMEM_SHARED((NP,), _f32),
          pltpu.SemaphoreType.DMA,
          pltpu.SemaphoreType.DMA,
          pltpu.SemaphoreType.DMA,
          pltpu.SemaphoreType.DMA,
      ],
  )
  def edge_pass(x_hbm, src_hbm, dst_hbm, z2d_hbm, z1d_hbm, on_hbm, outp_hbm,
                *refs):
    if with_deg:
      outd_hbm = refs[0]
      refs = refs[1:]
    sidx, didx, rowsa, rowsb, ones, acc, deg, gsa, gsb, ssa, ssb = refs
    c = lax.axis_index("c")
    s = lax.axis_index("s")
    wid = c * NS + s

    base = s * RPT
    pltpu.sync_copy(z2d_hbm, acc.at[pl.ds(base, RPT)])
    if with_deg:
      pltpu.sync_copy(z1d_hbm, deg.at[pl.ds(base, RPT)])
      pltpu.sync_copy(on_hbm, ones)
    plsc.subcore_barrier()

    def gloop(g, _):
      gb_ = wid * CPT + g * G
      pltpu.sync_copy(src_hbm.at[pl.ds(gb_, G)], sidx)
      pltpu.sync_copy(dst_hbm.at[pl.ds(gb_, G)], didx)
      pltpu.make_async_copy(x_hbm.at[sidx.at[0]], rowsa, gsa).start()
      pltpu.make_async_copy(x_hbm.at[sidx.at[1]], rowsb, gsb).start()

      def eloop(jj, _):
        j0 = 2 * jj
        j1 = j0 + 1
        pltpu.make_async_copy(x_hbm.at[sidx.at[j0]], rowsa, gsa).wait()
        sca = pltpu.make_async_copy(rowsa, acc.at[didx.at[j0]], ssa)
        sca.start(add=True)
        if with_deg:
          pltpu.sync_copy(ones, deg.at[didx.at[j0]], add=True)
        pltpu.make_async_copy(x_hbm.at[sidx.at[j1]], rowsb, gsb).wait()
        scb = pltpu.make_async_copy(rowsb, acc.at[didx.at[j1]], ssb)
        scb.start(add=True)
        if with_deg:
          pltpu.sync_copy(ones, deg.at[didx.at[j1]], add=True)
        sca.wait()

        @pl.when(j0 + 2 < G)
        def _():
          pltpu.make_async_copy(x_hbm.at[sidx.at[j0 + 2]], rowsa, gsa).start()
        scb.wait()

        @pl.when(j1 + 2 < G)
        def _():
          pltpu.make_async_copy(x_hbm.at[sidx.at[j1 + 2]], rowsb, gsb).start()
        return 0
      lax.fori_loop(0, G // 2, eloop, 0)
      return 0
    lax.fori_loop(0, CPT // G, gloop, 0)

    plsc.subcore_barrier()

    pltpu.sync_copy(acc.at[pl.ds(base, RPT)],
                    outp_hbm.at[pl.ds(c * NP + base, RPT)])
    if with_deg:
      pltpu.sync_copy(deg.at[pl.ds(base, RPT)],
                      outd_hbm.at[pl.ds(c * NP + base, RPT)])

  return edge_pass


def _make_dense1(NP, D, H, BM):
  NB = NP // BM

  def body(p0_ref, p1_ref, d0_ref, d1_ref, x_ref, wl_ref, wr_ref, b_ref,
           u_ref, pr_ref):
    d = d0_ref[0, 0, :] + d1_ref[0, 0, :]
    inv = 1.0 / jnp.maximum(d, 1.0)
    mean = (p0_ref[...] + p1_ref[...]) * inv[:, None]
    z = (jnp.dot(mean, wl_ref[...], preferred_element_type=_f32)
         + jnp.dot(x_ref[...], wr_ref[...], preferred_element_type=_f32)
         + b_ref[...])
    z = jnp.maximum(z, 0.0)
    pr_ref[...] = jax.lax.dot_general(
        u_ref[...], z, (((1,), (1,)), ((), ())),
        preferred_element_type=_f32)

  return pl.pallas_call(
      body,
      grid=(NB,),
      in_specs=[
          pl.BlockSpec((BM, D), lambda i: (i, 0)),
          pl.BlockSpec((BM, D), lambda i: (i + NB, 0)),
          pl.BlockSpec((1, 1, BM), lambda i: (i, 0, 0)),
          pl.BlockSpec((1, 1, BM), lambda i: (i + NB, 0, 0)),
          pl.BlockSpec((BM, D), lambda i: (i, 0)),
          pl.BlockSpec((D, H), lambda i: (0, 0)),
          pl.BlockSpec((D, H), lambda i: (0, 0)),
          pl.BlockSpec((1, H), lambda i: (0, 0)),
          pl.BlockSpec((8, H), lambda i: (0, 0)),
      ],
      out_specs=pl.BlockSpec((8, BM), lambda i: (0, i)),
      out_shape=jax.ShapeDtypeStruct((8, NP), _f32),
  )


def _make_edge_narrow(NP, CPT, GQ=16):
  RPT = NP // NS
  mesh = plsc.VectorSubcoreMesh(core_axis_name="c", subcore_axis_name="s",
                                num_cores=NC, num_subcores=NS)

  @functools.partial(
      pl.kernel,
      out_type=(
          jax.ShapeDtypeStruct((NC * NP,), _f32),
          jax.ShapeDtypeStruct((NC * NP,), _f32),
      ),
      mesh=mesh,
      scratch_types=[
          pltpu.VMEM((CPT, 128), jnp.int32),
          pltpu.VMEM((CPT, 128), jnp.int32),
          pltpu.VMEM((GQ * 128,), _f32),
          pltpu.VMEM((GQ * 128,), _f32),
          pltpu.VMEM_SHARED((NP,), _f32),
          pltpu.VMEM_SHARED((NP,), _f32),
          pltpu.VMEM_SHARED((NP,), _f32),
          pltpu.VMEM_SHARED((NP,), _f32),
          pltpu.SemaphoreType.DMA,
          pltpu.SemaphoreType.DMA,
          pltpu.SemaphoreType.DMA,
          pltpu.SemaphoreType.DMA,
      ],
  )
  def narrow(t_hbm, src_hbm, dst_hbm, z1d_hbm, outa_hbm, outb_hbm,
             si, di, bufa, bufb, ash, bsh, acca, accb, sma, smb, swa, swb):
    c = lax.axis_index("c")
    s = lax.axis_index("s")
    wid = c * NS + s
    base = s * RPT

    @pl.when(s == 0)
    def _():
      pltpu.sync_copy(t_hbm.at[0], ash)
      pltpu.sync_copy(t_hbm.at[1], bsh)
    pltpu.sync_copy(z1d_hbm, acca.at[pl.ds(base, RPT)])
    pltpu.sync_copy(z1d_hbm, accb.at[pl.ds(base, RPT)])
    pltpu.sync_copy(src_hbm.at[pl.ds(wid * CPT, CPT)], si)
    pltpu.sync_copy(dst_hbm.at[pl.ds(wid * CPT, CPT)], di)
    plsc.subcore_barrier()

    def loop(g, _):
      j0 = g * GQ

      @pl.when(g > 0)
      def _():
        for k in range(GQ):
          jp = j0 - GQ + k
          pltpu.make_async_copy(bufa.at[pl.ds(k * 128, 128)],
                                acca.at[di.at[jp]], swa).wait()
          pltpu.make_async_copy(bufb.at[pl.ds(k * 128, 128)],
                                accb.at[di.at[jp]], swb).wait()
      for k in range(GQ):
        pltpu.make_async_copy(ash.at[si.at[j0 + k]],
                              bufa.at[pl.ds(k * 128, 128)], sma).start()
        pltpu.make_async_copy(bsh.at[si.at[j0 + k]],
                              bufb.at[pl.ds(k * 128, 128)], smb).start()
      for k in range(GQ):
        pltpu.make_async_copy(ash.at[si.at[j0 + k]],
                              bufa.at[pl.ds(k * 128, 128)], sma).wait()
        pltpu.make_async_copy(bsh.at[si.at[j0 + k]],
                              bufb.at[pl.ds(k * 128, 128)], smb).wait()
      for k in range(GQ):
        pltpu.make_async_copy(bufa.at[pl.ds(k * 128, 128)],
                              acca.at[di.at[j0 + k]], swa).start(add=True)
        pltpu.make_async_copy(bufb.at[pl.ds(k * 128, 128)],
                              accb.at[di.at[j0 + k]], swb).start(add=True)
      return 0
    lax.fori_loop(0, CPT // GQ, loop, 0)

    for k in range(GQ):
      jp = CPT - GQ + k
      pltpu.make_async_copy(bufa.at[pl.ds(k * 128, 128)],
                            acca.at[di.at[jp]], swa).wait()
      pltpu.make_async_copy(bufb.at[pl.ds(k * 128, 128)],
                            accb.at[di.at[jp]], swb).wait()
    plsc.subcore_barrier()

    pltpu.sync_copy(acca.at[pl.ds(base, RPT)],
                    outa_hbm.at[pl.ds(c * NP + base, RPT)])
    pltpu.sync_copy(accb.at[pl.ds(base, RPT)],
                    outb_hbm.at[pl.ds(c * NP + base, RPT)])

  return narrow


def _make_decode(NP, TE, GQ=14):
  EPT = TE // NW
  CQ = EPT // 128
  RPT = NP // NS
  mesh = plsc.VectorSubcoreMesh(core_axis_name="c", subcore_axis_name="s",
                                num_cores=NC, num_subcores=NS)

  @functools.partial(
      pl.kernel,
      out_type=jax.ShapeDtypeStruct((TE,), _f32),
      mesh=mesh,
      scratch_types=[
          pltpu.VMEM((CQ, 128), jnp.int32),
          pltpu.VMEM((CQ, 128), jnp.int32),
          pltpu.VMEM((GQ * 128,), _f32),
          pltpu.VMEM((GQ * 128,), _f32),
          pltpu.VMEM((EPT,), _f32),
          pltpu.VMEM((RPT,), _f32),
          pltpu.VMEM((RPT,), _f32),
          pltpu.VMEM((RPT,), _f32),
          pltpu.VMEM((RPT,), _f32),
          pltpu.VMEM((RPT,), _f32),
          pltpu.VMEM((RPT,), _f32),
          pltpu.VMEM((RPT,), _f32),
          pltpu.VMEM((RPT,), _f32),
          pltpu.VMEM((RPT,), _f32),
          pltpu.VMEM((RPT,), _f32),
          pltpu.VMEM_SHARED((NP,), _f32),
          pltpu.VMEM_SHARED((NP,), _f32),
          pltpu.SemaphoreType.DMA,
          pltpu.SemaphoreType.DMA,
      ],
  )
  def decode(pa_hbm, pb_hbm, dg_hbm, ga_hbm, gb_hbm, s_hbm, d_hbm, out_hbm,
             si, di, bufa, bufb, ov, a0, a1, b0, b1, dd0, dd1, gga, ggb,
             ta, tb, ash, bsh, sma, smb):
    c = lax.axis_index("c")
    s = lax.axis_index("s")
    wid = c * NS + s
    base = s * RPT
    pltpu.sync_copy(pa_hbm.at[pl.ds(base, RPT)], a0)
    pltpu.sync_copy(pa_hbm.at[pl.ds(NP + base, RPT)], a1)
    pltpu.sync_copy(pb_hbm.at[pl.ds(base, RPT)], b0)
    pltpu.sync_copy(pb_hbm.at[pl.ds(NP + base, RPT)], b1)
    pltpu.sync_copy(dg_hbm.at[pl.ds(base, RPT)], dd0)
    pltpu.sync_copy(dg_hbm.at[pl.ds(NP + base, RPT)], dd1)
    pltpu.sync_copy(ga_hbm.at[pl.ds(base, RPT)], gga)
    pltpu.sync_copy(gb_hbm.at[pl.ds(base, RPT)], ggb)
    pltpu.sync_copy(s_hbm.at[pl.ds(wid * CQ, CQ)], si)
    pltpu.sync_copy(d_hbm.at[pl.ds(wid * CQ, CQ)], di)

    def build(i, _):
      o = i * 16
      sl = pl.ds(o, 16)
      inv = 1.0 / jnp.maximum(dd0[sl] + dd1[sl], 1.0)
      ta[sl] = (a0[sl] + a1[sl]) * inv + gga[sl]
      tb[sl] = (b0[sl] + b1[sl]) * inv + ggb[sl]
      return 0
    lax.fori_loop(0, RPT // 16, build, 0)
    pltpu.sync_copy(ta, ash.at[pl.ds(base, RPT)])
    pltpu.sync_copy(tb, bsh.at[pl.ds(base, RPT)])
    plsc.subcore_barrier()

    def loop(g, _):
      j0 = g * GQ
      for k in range(GQ):
        pltpu.make_async_copy(ash.at[si.at[j0 + k]],
                              bufa.at[pl.ds(k * 128, 128)], sma).start()
        pltpu.make_async_copy(bsh.at[di.at[j0 + k]],
                              bufb.at[pl.ds(k * 128, 128)], smb).start()
      for k in range(GQ):
        pltpu.make_async_copy(ash.at[si.at[j0 + k]],
                              bufa.at[pl.ds(k * 128, 128)], sma).wait()
        pltpu.make_async_copy(bsh.at[di.at[j0 + k]],
                              bufb.at[pl.ds(k * 128, 128)], smb).wait()
      for k in range(GQ * 128 // 16):
        o = k * 16
        ov[pl.ds(j0 * 128 + o, 16)] = (bufa[pl.ds(o, 16)]
                                       + bufb[pl.ds(o, 16)])
      return 0
    lax.fori_loop(0, CQ // GQ, loop, 0)

    pltpu.sync_copy(ov, out_hbm.at[pl.ds(wid * EPT, EPT)])

  return decode


def _pad_to(v, m):
  return ((v + m - 1) // m) * m


def kernel(x, edge_index, edge_weight, pos_edge_index, neg_edge_index,
           W1l, b1l, W1r, W2l, b2l, W2r, Wp, bp):
  N, D = x.shape
  H = W1l.shape[0]
  E = edge_index.shape[1]
  PE = pos_edge_index.shape[1]
  NE = neg_edge_index.shape[1]

  BM = 512
  NP = _pad_to(N, max(BM, NS * 128))
  EP = _pad_to(E, NW * 256)
  CPT = EP // (128 * NW)

  xp = jnp.zeros((NP, D), _f32).at[:N].set(x)

  npad = EP - E
  pad_src = (jnp.arange(npad, dtype=jnp.int32) * 97) % N
  pad_dst = N + (jnp.arange(npad, dtype=jnp.int32) % (NP - N))
  src = jnp.concatenate([edge_index[0], pad_src]).reshape(EP // 128, 128)
  dst = jnp.concatenate([edge_index[1], pad_dst]).reshape(EP // 128, 128)

  W1lT = W1l.T
  W1rT = W1r.T
  b1 = b1l.reshape(1, H)

  wa = Wp[0, :H]
  wb = Wp[0, H:]
  U8 = (jnp.zeros((8, H), _f32)
        .at[0].set(W2l.T @ wa).at[1].set(W2l.T @ wb)
        .at[2].set(W2r.T @ wa).at[3].set(W2r.T @ wb))
  cvec8 = (jnp.zeros((1, 8), _f32)
           .at[0, 0].set(b2l @ wa + bp[0])
           .at[0, 1].set(b2l @ wb))

  PP = _pad_to(PE, 128)
  TE = _pad_to(PP + NE, NW * 8 * 128)
  fill = (jnp.arange(TE, dtype=jnp.int32) * 89) % N
  qsrc = fill.at[:PE].set(pos_edge_index[0]).at[PP:PP + NE].set(neg_edge_index[0])
  qdst = fill.at[:PE].set(pos_edge_index[1]).at[PP:PP + NE].set(neg_edge_index[1])
  qsrc = qsrc.reshape(TE // 128, 128)
  qdst = qdst.reshape(TE // 128, 128)

  zr2 = jnp.zeros((NP // NS, D), _f32)
  zr1 = jnp.zeros((NP // NS,), _f32)
  on1 = jnp.ones((128,), _f32)

  edge_pass = _make_edge_pass(NP, D, CPT, with_deg=True)
  edge_narrow = _make_edge_narrow(NP, CPT)
  dense1 = _make_dense1(NP, D, H, BM)
  decode = _make_decode(NP, TE)
  NB = NP // BM

  p, dg = edge_pass(xp, src, dst, zr2, zr1, on1)
  d3 = dg.reshape(NC * NB, 1, BM)
  proj8 = dense1(p, p, d3, d3, xp, W1lT, W1rT, b1, U8)

  pa, pb = edge_narrow(proj8, src, dst, zr1)

  ga = proj8[2] + cvec8[0, 0]
  gb = proj8[3] + cvec8[0, 1]
  dec = decode(pa, pb, dg, ga, gb, qsrc, qdst)
  pos = dec[:PE]
  neg = dec[PP:PP + NE]
  return (pos, neg)
